# baseline (speedup 1.0000x reference)
import functools
import jax
import jax.numpy as jnp
from jax import lax
from jax.experimental import pallas as pl
from jax.experimental.pallas import tpu as pltpu

K = 20
KP = 24
B, N, IN_DIMS = 16, 1024, 3
FEATURE_DIMS = [64, 64, 128, 256]
BN = B * N


def _leaky(x):
    return jnp.where(x >= 0, x, 0.2 * x)


def _bf(x):
    return x.astype(jnp.bfloat16)


def _dist_body(h_ref, dist_ref):
    h = h_ref[...]
    sq = jnp.sum(h * h, axis=1, keepdims=True)
    hb = _bf(h)
    mm = lax.dot_general(hb, hb, (((1,), (1,)), ((), ())),
                         preferred_element_type=jnp.float32)
    dist_ref[...] = (sq + sq.reshape(1, N)) - 2.0 * mm


def _dist(h2d, C):
    return pl.pallas_call(
        _dist_body,
        grid=(B,),
        in_specs=[pl.BlockSpec((N, C), lambda b: (b, 0))],
        out_specs=pl.BlockSpec((N, N), lambda b: (b, 0)),
        out_shape=jax.ShapeDtypeStruct((BN, N), jnp.float32),
    )(h2d)


def _edge_body(nbr_ref, h_ref, tw_ref, pw_ref, bias_ref,
               emax_ref, emin_ref, stats_ref, acc_ref, DSTBLK, Cin, Cout):
    i = pl.program_id(0)
    nbr = nbr_ref[...].reshape(DSTBLK, KP, Cin)
    h = h_ref[...]
    diff = h[:, None, :] - nbr
    db = _bf(diff).reshape(DSTBLK * KP, Cin)
    nb = _bf(nbr).reshape(DSTBLK * KP, Cin)
    e = (jnp.dot(db, tw_ref[...], preferred_element_type=jnp.float32)
         + jnp.dot(nb, pw_ref[...], preferred_element_type=jnp.float32)
         + bias_ref[...])
    e = e.reshape(DSTBLK, KP, Cout)[:, :K, :]
    emax_ref[...] = jnp.max(e, axis=1)
    emin_ref[...] = jnp.min(e, axis=1)
    s = jnp.sum(e.reshape(DSTBLK * K, Cout), axis=0, keepdims=True)
    s2 = jnp.sum((e * e).reshape(DSTBLK * K, Cout), axis=0, keepdims=True)

    @pl.when(i == 0)
    def _():
        acc_ref[...] = jnp.zeros_like(acc_ref)

    acc_ref[0:1, :] += s
    acc_ref[1:2, :] += s2
    stats_ref[...] = acc_ref[...]


def _edge(nbr2d, h2d, tw, pw, bias, Cin, Cout):
    DSTBLK = 128
    body = functools.partial(_edge_body, DSTBLK=DSTBLK, Cin=Cin, Cout=Cout)
    return pl.pallas_call(
        body,
        grid=(BN // DSTBLK,),
        in_specs=[
            pl.BlockSpec((DSTBLK * KP, Cin), lambda i: (i, 0)),
            pl.BlockSpec((DSTBLK, Cin), lambda i: (i, 0)),
            pl.BlockSpec((Cin, Cout), lambda i: (0, 0)),
            pl.BlockSpec((Cin, Cout), lambda i: (0, 0)),
            pl.BlockSpec((1, Cout), lambda i: (0, 0)),
        ],
        out_specs=[
            pl.BlockSpec((DSTBLK, Cout), lambda i: (i, 0)),
            pl.BlockSpec((DSTBLK, Cout), lambda i: (i, 0)),
            pl.BlockSpec((8, Cout), lambda i: (0, 0)),
        ],
        out_shape=[
            jax.ShapeDtypeStruct((BN, Cout), jnp.float32),
            jax.ShapeDtypeStruct((BN, Cout), jnp.float32),
            jax.ShapeDtypeStruct((8, Cout), jnp.float32),
        ],
        scratch_shapes=[pltpu.VMEM((8, Cout), jnp.float32)],
    )(nbr2d, h2d, _bf(tw), _bf(pw), bias)


def _bnact_body(emax_ref, emin_ref, stats_ref, g_ref, bta_ref, out_ref):
    cnt = float(BN * K)
    mean = stats_ref[0:1, :] / cnt
    var = stats_ref[1:2, :] / cnt - mean * mean
    scale = g_ref[...] * lax.rsqrt(var + 1e-5)
    red = jnp.where(scale >= 0, emax_ref[...], emin_ref[...])
    out_ref[...] = _leaky(scale * (red - mean) + bta_ref[...])


def _bnact(emax, emin, stats, g, bta, Cout):
    RB = 1024
    return pl.pallas_call(
        _bnact_body,
        grid=(BN // RB,),
        in_specs=[
            pl.BlockSpec((RB, Cout), lambda i: (i, 0)),
            pl.BlockSpec((RB, Cout), lambda i: (i, 0)),
            pl.BlockSpec((8, Cout), lambda i: (0, 0)),
            pl.BlockSpec((1, Cout), lambda i: (0, 0)),
            pl.BlockSpec((1, Cout), lambda i: (0, 0)),
        ],
        out_specs=pl.BlockSpec((RB, Cout), lambda i: (i, 0)),
        out_shape=jax.ShapeDtypeStruct((BN, Cout), jnp.float32),
    )(emax, emin, stats, g.reshape(1, -1), bta.reshape(1, -1))


def _proj_body(h0_ref, h1_ref, h2_ref, h3_ref, w_ref, b_ref, out_ref):
    hcat = jnp.concatenate([h0_ref[...], h1_ref[...], h2_ref[...],
                            h3_ref[...]], axis=1)
    p = (jnp.dot(_bf(hcat), w_ref[...], preferred_element_type=jnp.float32)
         + b_ref[...])
    mx = jnp.max(p, axis=0, keepdims=True)
    mean = jnp.sum(p, axis=0, keepdims=True) / float(N)
    out_ref[0, 0:1, :] = mx
    out_ref[0, 1:2, :] = mean


def _proj_pool(hs, w, b):
    E = w.shape[1]
    out = pl.pallas_call(
        _proj_body,
        grid=(B,),
        in_specs=[pl.BlockSpec((1024, c), lambda i: (i, 0))
                  for c in FEATURE_DIMS]
        + [pl.BlockSpec((sum(FEATURE_DIMS), E), lambda i: (0, 0)),
           pl.BlockSpec((1, E), lambda i: (0, 0))],
        out_specs=pl.BlockSpec((1, 2, E), lambda i: (i, 0, 0)),
        out_shape=jax.ShapeDtypeStruct((B, 2, E), jnp.float32),
    )(*hs, _bf(w), b.reshape(1, -1))
    return jnp.concatenate([out[:, 0, :], out[:, 1, :]], axis=1)


def _head_body(h_ref, w0_ref, b0_ref, w1_ref, b1_ref, w2_ref, b2_ref, o_ref):
    h = h_ref[...]
    h = _leaky(jnp.dot(_bf(h), w0_ref[...],
                       preferred_element_type=jnp.float32) + b0_ref[...])
    h = _leaky(jnp.dot(_bf(h), w1_ref[...],
                       preferred_element_type=jnp.float32) + b1_ref[...])
    o_ref[...] = (jnp.dot(_bf(h), w2_ref[...],
                          preferred_element_type=jnp.float32) + b2_ref[...])


def _head(h2, w0, b0, w1, b1, w2, b2):
    return pl.pallas_call(
        _head_body,
        out_shape=jax.ShapeDtypeStruct((B, w2.shape[1]), jnp.float32),
    )(h2, _bf(w0), b0.reshape(1, -1), _bf(w1), b1.reshape(1, -1),
      _bf(w2), b2.reshape(1, -1))


def _topk_idx(dist):
    _, idx = lax.top_k(-dist.reshape(B, N, N), K)
    idx24 = jnp.concatenate(
        [idx, jnp.broadcast_to(idx[..., K - 1:K], (B, N, KP - K))], axis=-1)
    return idx24.reshape(BN, KP).astype(jnp.int32)


def _gather_nbr(h2d, idx24, C):
    h3 = h2d.reshape(B, N, C)
    i3 = idx24.reshape(B, N, KP)
    bidx = jnp.arange(B)[:, None, None]
    return h3[bidx, i3].reshape(BN * KP, C)


def _edge_conv_layer(h2d, Cin, tw, tb, pw, pb, g, bta):
    Cout = tw.shape[1]
    dist = _dist(h2d, Cin)
    idx24 = _topk_idx(dist)
    nbr = _gather_nbr(h2d, idx24, Cin)
    emax, emin, stats = _edge(nbr, h2d, tw, pw, (tb + pb).reshape(1, -1),
                              Cin, Cout)
    return _bnact(emax, emin, stats, g, bta, Cout)


def kernel(x,
           theta_W0, theta_b0, phi_W0, phi_b0, bn_g0, bn_b0,
           theta_W1, theta_b1, phi_W1, phi_b1, bn_g1, bn_b1,
           theta_W2, theta_b2, phi_W2, phi_b2, bn_g2, bn_b2,
           theta_W3, theta_b3, phi_W3, phi_b3, bn_g3, bn_b3,
           proj_W, proj_b,
           emb_W0, emb_b0, emb_W1, emb_b1,
           out_W, out_b):
    inp = dict(locals())
    h = jnp.pad(x.reshape(BN, IN_DIMS), ((0, 0), (0, 5)))
    tw0 = jnp.pad(theta_W0, ((0, 5), (0, 0)))
    pw0 = jnp.pad(phi_W0, ((0, 5), (0, 0)))
    Cin = 8
    hs = []
    for i in range(len(FEATURE_DIMS)):
        tw = tw0 if i == 0 else inp[f"theta_W{i}"]
        pw = pw0 if i == 0 else inp[f"phi_W{i}"]
        h = _edge_conv_layer(h, Cin, tw, inp[f"theta_b{i}"],
                             pw, inp[f"phi_b{i}"],
                             inp[f"bn_g{i}"], inp[f"bn_b{i}"])
        Cin = FEATURE_DIMS[i]
        hs.append(h)
    h2 = _proj_pool(hs, proj_W, proj_b)
    return _head(h2, emb_W0, emb_b0, emb_W1, emb_b1, out_W, out_b)


# speedup vs baseline: 4.2754x; 4.2754x over previous
import functools
import jax
import jax.numpy as jnp
from jax import lax
from jax.experimental import pallas as pl
from jax.experimental.pallas import tpu as pltpu
from jax.experimental.pallas import tpu_sc as plsc

K = 20
KP = 24
B, N, IN_DIMS = 16, 1024, 3
FEATURE_DIMS = [64, 64, 128, 256]
BN = B * N


def _leaky(x):
    return jnp.where(x >= 0, x, 0.2 * x)


def _bf(x):
    return x.astype(jnp.bfloat16)


def _dist_body(h_ref, dist_ref):
    h = h_ref[...]
    sq = jnp.sum(h * h, axis=1, keepdims=True)
    hb = _bf(h)
    mm = lax.dot_general(hb, hb, (((1,), (1,)), ((), ())),
                         preferred_element_type=jnp.float32)
    dist_ref[...] = (sq + sq.reshape(1, N)) - 2.0 * mm


def _dist(h2d, C):
    return pl.pallas_call(
        _dist_body,
        grid=(B,),
        in_specs=[pl.BlockSpec((N, C), lambda b: (b, 0))],
        out_specs=pl.BlockSpec((N, N), lambda b: (b, 0)),
        out_shape=jax.ShapeDtypeStruct((BN, N), jnp.float32),
    )(h2d)


def _edge_body(nbr_ref, h_ref, tw_ref, pw_ref, bias_ref,
               emax_ref, emin_ref, stats_ref, acc_ref, DSTBLK, Cin, Cout):
    i = pl.program_id(0)
    nbr = nbr_ref[...].reshape(DSTBLK, KP, Cin)
    h = h_ref[...]
    diff = h[:, None, :] - nbr
    db = _bf(diff).reshape(DSTBLK * KP, Cin)
    nb = _bf(nbr).reshape(DSTBLK * KP, Cin)
    e = (jnp.dot(db, tw_ref[...], preferred_element_type=jnp.float32)
         + jnp.dot(nb, pw_ref[...], preferred_element_type=jnp.float32)
         + bias_ref[...])
    e = e.reshape(DSTBLK, KP, Cout)[:, :K, :]
    emax_ref[...] = jnp.max(e, axis=1)
    emin_ref[...] = jnp.min(e, axis=1)
    s = jnp.sum(e.reshape(DSTBLK * K, Cout), axis=0, keepdims=True)
    s2 = jnp.sum((e * e).reshape(DSTBLK * K, Cout), axis=0, keepdims=True)

    @pl.when(i == 0)
    def _():
        acc_ref[...] = jnp.zeros_like(acc_ref)

    acc_ref[0:1, :] += s
    acc_ref[1:2, :] += s2
    stats_ref[...] = acc_ref[...]


def _edge(nbr2d, h2d, tw, pw, bias, Cin, Cout):
    DSTBLK = 128
    body = functools.partial(_edge_body, DSTBLK=DSTBLK, Cin=Cin, Cout=Cout)
    return pl.pallas_call(
        body,
        grid=(BN // DSTBLK,),
        in_specs=[
            pl.BlockSpec((DSTBLK * KP, Cin), lambda i: (i, 0)),
            pl.BlockSpec((DSTBLK, Cin), lambda i: (i, 0)),
            pl.BlockSpec((Cin, Cout), lambda i: (0, 0)),
            pl.BlockSpec((Cin, Cout), lambda i: (0, 0)),
            pl.BlockSpec((1, Cout), lambda i: (0, 0)),
        ],
        out_specs=[
            pl.BlockSpec((DSTBLK, Cout), lambda i: (i, 0)),
            pl.BlockSpec((DSTBLK, Cout), lambda i: (i, 0)),
            pl.BlockSpec((8, Cout), lambda i: (0, 0)),
        ],
        out_shape=[
            jax.ShapeDtypeStruct((BN, Cout), jnp.float32),
            jax.ShapeDtypeStruct((BN, Cout), jnp.float32),
            jax.ShapeDtypeStruct((8, Cout), jnp.float32),
        ],
        scratch_shapes=[pltpu.VMEM((8, Cout), jnp.float32)],
    )(nbr2d, h2d, _bf(tw), _bf(pw), bias)


def _bnact_body(emax_ref, emin_ref, stats_ref, g_ref, bta_ref, out_ref):
    cnt = float(BN * K)
    mean = stats_ref[0:1, :] / cnt
    var = stats_ref[1:2, :] / cnt - mean * mean
    scale = g_ref[...] * lax.rsqrt(var + 1e-5)
    red = jnp.where(scale >= 0, emax_ref[...], emin_ref[...])
    out_ref[...] = _leaky(scale * (red - mean) + bta_ref[...])


def _bnact(emax, emin, stats, g, bta, Cout):
    RB = 1024
    return pl.pallas_call(
        _bnact_body,
        grid=(BN // RB,),
        in_specs=[
            pl.BlockSpec((RB, Cout), lambda i: (i, 0)),
            pl.BlockSpec((RB, Cout), lambda i: (i, 0)),
            pl.BlockSpec((8, Cout), lambda i: (0, 0)),
            pl.BlockSpec((1, Cout), lambda i: (0, 0)),
            pl.BlockSpec((1, Cout), lambda i: (0, 0)),
        ],
        out_specs=pl.BlockSpec((RB, Cout), lambda i: (i, 0)),
        out_shape=jax.ShapeDtypeStruct((BN, Cout), jnp.float32),
    )(emax, emin, stats, g.reshape(1, -1), bta.reshape(1, -1))


def _proj_body(h0_ref, h1_ref, h2_ref, h3_ref, w_ref, b_ref, out_ref):
    hcat = jnp.concatenate([h0_ref[...], h1_ref[...], h2_ref[...],
                            h3_ref[...]], axis=1)
    p = (jnp.dot(_bf(hcat), w_ref[...], preferred_element_type=jnp.float32)
         + b_ref[...])
    mx = jnp.max(p, axis=0, keepdims=True)
    mean = jnp.sum(p, axis=0, keepdims=True) / float(N)
    out_ref[0, 0:1, :] = mx
    out_ref[0, 1:2, :] = mean


def _proj_pool(hs, w, b):
    E = w.shape[1]
    out = pl.pallas_call(
        _proj_body,
        grid=(B,),
        in_specs=[pl.BlockSpec((1024, c), lambda i: (i, 0))
                  for c in FEATURE_DIMS]
        + [pl.BlockSpec((sum(FEATURE_DIMS), E), lambda i: (0, 0)),
           pl.BlockSpec((1, E), lambda i: (0, 0))],
        out_specs=pl.BlockSpec((1, 2, E), lambda i: (i, 0, 0)),
        out_shape=jax.ShapeDtypeStruct((B, 2, E), jnp.float32),
    )(*hs, _bf(w), b.reshape(1, -1))
    return jnp.concatenate([out[:, 0, :], out[:, 1, :]], axis=1)


def _head_body(h_ref, w0_ref, b0_ref, w1_ref, b1_ref, w2_ref, b2_ref, o_ref):
    h = h_ref[...]
    h = _leaky(jnp.dot(_bf(h), w0_ref[...],
                       preferred_element_type=jnp.float32) + b0_ref[...])
    h = _leaky(jnp.dot(_bf(h), w1_ref[...],
                       preferred_element_type=jnp.float32) + b1_ref[...])
    o_ref[...] = (jnp.dot(_bf(h), w2_ref[...],
                          preferred_element_type=jnp.float32) + b2_ref[...])


def _head(h2, w0, b0, w1, b1, w2, b2):
    return pl.pallas_call(
        _head_body,
        out_shape=jax.ShapeDtypeStruct((B, w2.shape[1]), jnp.float32),
    )(h2, _bf(w0), b0.reshape(1, -1), _bf(w1), b1.reshape(1, -1),
      _bf(w2), b2.reshape(1, -1))


def _topk_idx(dist):
    NW = 32
    RG = 8
    per_w = BN // NW
    n_g = per_w // RG
    INF = jnp.float32(jnp.inf)
    mesh = plsc.VectorSubcoreMesh(core_axis_name="c", subcore_axis_name="s")

    def _scal(v, lane_i):
        return lax.squeeze(lax.slice(v, (lane_i,), (lane_i + 1,)), (0,))

    def _cnt(msk):
        return _scal(plsc.all_reduce_population_count(msk), 0)

    _DN = lax.GatherDimensionNumbers(offset_dims=(), collapsed_slice_dims=(0,),
                                     start_index_map=(0,))

    def _perm(v, idxvec):
        return lax.gather(v, idxvec[:, None], _DN, (1,),
                          mode=lax.GatherScatterMode.PROMISE_IN_BOUNDS)

    @functools.partial(
        pl.kernel, mesh=mesh,
        compiler_params=pltpu.CompilerParams(use_tc_tiling_on_sc=False, needs_layout_passes=False),
        out_type=jax.ShapeDtypeStruct((BN * KP,), jnp.int32),
        scratch_types=[
            pltpu.VMEM((RG * N,), jnp.float32),
            pltpu.VMEM((N + 16,), jnp.float32),
            pltpu.VMEM((N + 16,), jnp.int32),
            pltpu.VMEM((RG * KP + 16,), jnp.int32),
        ],
    )
    def t(dist_hbm, out_hbm, rowbuf, cand_v, cand_i, outb):
        wid = lax.axis_index("s") * 2 + lax.axis_index("c")
        lane = lax.iota(jnp.int32, 16)

        def splat_min(v):
            for sh in (8, 4, 2, 1):
                v = jnp.minimum(v, _perm(v, jnp.bitwise_xor(lane, sh)))
            return v

        def splat_max(v):
            for sh in (8, 4, 2, 1):
                v = jnp.maximum(v, _perm(v, jnp.bitwise_xor(lane, sh)))
            return v

        def row_body(r, base_row):
            rb = r * N
            row_gid = base_row + r
            m0 = rowbuf[pl.ds(rb, 16)]
            m1 = rowbuf[pl.ds(rb + 512, 16)]
            for j in range(1, 32):
                m0 = jnp.minimum(m0, rowbuf[pl.ds(rb + j * 16, 16)])
                m1 = jnp.minimum(m1, rowbuf[pl.ds(rb + 512 + j * 16, 16)])
            thv = splat_max(jnp.maximum(m0, m1))
            ptr = jnp.int32(0)
            for j in range(64):
                v = rowbuf[pl.ds(rb + j * 16, 16)]
                msk = v <= thv
                plsc.store_compressed(cand_v.at[pl.ds(ptr, 16)], v, mask=msk)
                plsc.store_compressed(cand_i.at[pl.ds(ptr, 16)],
                                      lane + (j * 16), mask=msk)
                ptr = ptr + _cnt(msk)
            cand_v[pl.ds(ptr, 16)] = jnp.full((16,), INF)
            nt = (ptr + 15) // 16

            def extract(_, carry):
                tot, m1c, tau = carry

                def mpass(tc, mv):
                    v = cand_v[pl.ds(tc * 16, 16)]
                    return jnp.minimum(mv, jnp.where(v > tau, v, INF))

                m = splat_min(lax.fori_loop(0, nt, mpass,
                                            jnp.full((16,), INF)))

                def cpass(tc, cc):
                    v = cand_v[pl.ds(tc * 16, 16)]
                    return cc + _cnt(v == m)

                c = lax.fori_loop(0, nt, cpass, jnp.int32(0))
                live = tot < K
                m1c = jnp.where(live, tot, m1c)
                tau = jnp.where(live, m, tau)
                tot = jnp.where(live, tot + c, tot)
                return tot, m1c, tau

            _, m1c, tau = lax.fori_loop(
                0, K, extract,
                (jnp.int32(0), jnp.int32(0), jnp.full((16,), -INF)))
            m2v = jnp.full((16,), K - m1c, jnp.int32)

            gbase = jnp.full((16,), (row_gid // N) * N, jnp.int32)
            pad = jnp.full((16,), row_gid, jnp.int32)
            outb[pl.ds(r * KP, 16)] = pad
            outb[pl.ds(r * KP + 8, 16)] = pad

            def emit(tc, carry):
                optr, eqseen = carry
                v = cand_v[pl.ds(tc * 16, 16)]
                ci = cand_i[pl.ds(tc * 16, 16)]
                lt = v < tau
                eq = v == tau
                eqc = eq.astype(jnp.int32)
                for sh in (1, 2, 4, 8):
                    shifted = _perm(eqc, jnp.maximum(lane - sh, 0))
                    eqc = eqc + jnp.where(lane >= sh, shifted, 0)
                eqc = eqc + eqseen
                em = lt | (eq & (eqc <= m2v))
                plsc.store_compressed(outb.at[pl.ds(r * KP + optr, 16)],
                                      ci + gbase, mask=em)
                return (optr + _cnt(em),
                        eqseen + plsc.all_reduce_population_count(eq))

            lax.fori_loop(0, nt, emit,
                          (jnp.int32(0), jnp.zeros((16,), jnp.int32)))
            return base_row

        def g_body(g, _):
            base_row = wid * per_w + g * RG
            pltpu.sync_copy(dist_hbm.at[pl.ds(base_row * N, RG * N)], rowbuf)
            lax.fori_loop(0, RG, row_body, base_row)
            pltpu.sync_copy(outb.at[pl.ds(0, RG * KP)],
                            out_hbm.at[pl.ds(base_row * KP, RG * KP)])
            return 0

        lax.fori_loop(0, n_g, g_body, 0)

    return t(dist.reshape(BN * N))


def _gather_nbr(h2d, idx_flat, C):
    NW = 32
    CHUNK = 128
    ROWS = BN * KP
    per_w = ROWS // NW
    n_it = per_w // CHUNK
    mesh = plsc.VectorSubcoreMesh(core_axis_name="c", subcore_axis_name="s")

    @functools.partial(
        pl.kernel, mesh=mesh,
        compiler_params=pltpu.CompilerParams(use_tc_tiling_on_sc=False, needs_layout_passes=False),
        out_type=jax.ShapeDtypeStruct((ROWS, C), jnp.float32),
        scratch_types=[
            pltpu.VMEM((CHUNK,), jnp.int32),
            pltpu.VMEM((CHUNK, C), jnp.float32),
            pltpu.SemaphoreType.DMA,
        ],
    )
    def g(idx_hbm, table_hbm, out_hbm, idx_v, rows_v, sem):
        wid = lax.axis_index("s") * 2 + lax.axis_index("c")

        def body(i, _):
            base = wid * per_w + i * CHUNK
            pltpu.sync_copy(idx_hbm.at[pl.ds(base, CHUNK)], idx_v)
            pltpu.async_copy(table_hbm.at[idx_v], rows_v, sem).wait()
            pltpu.sync_copy(rows_v, out_hbm.at[pl.ds(base, CHUNK)])
            return 0

        lax.fori_loop(0, n_it, body, 0)

    return g(idx_flat, h2d)


def _edge_conv_layer(h2d, Cin, tw, tb, pw, pb, g, bta):
    Cout = tw.shape[1]
    dist = _dist(h2d, Cin)
    idx24 = _topk_idx(dist)
    nbr = _gather_nbr(h2d, idx24, Cin)
    emax, emin, stats = _edge(nbr, h2d, tw, pw, (tb + pb).reshape(1, -1),
                              Cin, Cout)
    return _bnact(emax, emin, stats, g, bta, Cout)


def kernel(x,
           theta_W0, theta_b0, phi_W0, phi_b0, bn_g0, bn_b0,
           theta_W1, theta_b1, phi_W1, phi_b1, bn_g1, bn_b1,
           theta_W2, theta_b2, phi_W2, phi_b2, bn_g2, bn_b2,
           theta_W3, theta_b3, phi_W3, phi_b3, bn_g3, bn_b3,
           proj_W, proj_b,
           emb_W0, emb_b0, emb_W1, emb_b1,
           out_W, out_b):
    inp = dict(locals())
    h = jnp.pad(x.reshape(BN, IN_DIMS), ((0, 0), (0, 5)))
    tw0 = jnp.pad(theta_W0, ((0, 5), (0, 0)))
    pw0 = jnp.pad(phi_W0, ((0, 5), (0, 0)))
    Cin = 8
    hs = []
    for i in range(len(FEATURE_DIMS)):
        tw = tw0 if i == 0 else inp[f"theta_W{i}"]
        pw = pw0 if i == 0 else inp[f"phi_W{i}"]
        h = _edge_conv_layer(h, Cin, tw, inp[f"theta_b{i}"],
                             pw, inp[f"phi_b{i}"],
                             inp[f"bn_g{i}"], inp[f"bn_b{i}"])
        Cin = FEATURE_DIMS[i]
        hs.append(h)
    h2 = _proj_pool(hs, proj_W, proj_b)
    return _head(h2, emb_W0, emb_b0, emb_W1, emb_b1, out_W, out_b)


# speedup vs baseline: 7.6289x; 1.7844x over previous
import functools
import jax
import jax.numpy as jnp
from jax import lax
from jax.experimental import pallas as pl
from jax.experimental.pallas import tpu as pltpu
from jax.experimental.pallas import tpu_sc as plsc

K = 20
KP = 24
B, N, IN_DIMS = 16, 1024, 3
FEATURE_DIMS = [64, 64, 128, 256]
BN = B * N


def _leaky(x):
    return jnp.where(x >= 0, x, 0.2 * x)


def _bf(x):
    return x.astype(jnp.bfloat16)


def _dist_body(h_ref, dist_ref):
    h = h_ref[...]
    sq = jnp.sum(h * h, axis=1, keepdims=True)
    hb = _bf(h)
    mm = lax.dot_general(hb, hb, (((1,), (1,)), ((), ())),
                         preferred_element_type=jnp.float32)
    dist_ref[...] = (sq + sq.reshape(1, N)) - 2.0 * mm


def _dist(h2d, C):
    return pl.pallas_call(
        _dist_body,
        grid=(B,),
        in_specs=[pl.BlockSpec((N, C), lambda b: (b, 0))],
        out_specs=pl.BlockSpec((N, N), lambda b: (b, 0)),
        out_shape=jax.ShapeDtypeStruct((BN, N), jnp.float32),
    )(h2d)


def _edge_body(nbr_ref, h_ref, tw_ref, pw_ref, bias_ref,
               emax_ref, emin_ref, stats_ref, acc_ref, DSTBLK, Cin, Cout):
    i = pl.program_id(0)
    nbr = nbr_ref[...].reshape(DSTBLK, KP, Cin)
    h = h_ref[...]
    diff = h[:, None, :] - nbr
    db = _bf(diff).reshape(DSTBLK * KP, Cin)
    nb = _bf(nbr).reshape(DSTBLK * KP, Cin)
    e = (jnp.dot(db, tw_ref[...], preferred_element_type=jnp.float32)
         + jnp.dot(nb, pw_ref[...], preferred_element_type=jnp.float32)
         + bias_ref[...])
    e = e.reshape(DSTBLK, KP, Cout)[:, :K, :]
    emax_ref[...] = jnp.max(e, axis=1)
    emin_ref[...] = jnp.min(e, axis=1)
    s = jnp.sum(e.reshape(DSTBLK * K, Cout), axis=0, keepdims=True)
    s2 = jnp.sum((e * e).reshape(DSTBLK * K, Cout), axis=0, keepdims=True)

    @pl.when(i == 0)
    def _():
        acc_ref[...] = jnp.zeros_like(acc_ref)

    acc_ref[0:1, :] += s
    acc_ref[1:2, :] += s2
    stats_ref[...] = acc_ref[...]


def _edge(nbr2d, h2d, tw, pw, bias, Cin, Cout):
    DSTBLK = 128
    body = functools.partial(_edge_body, DSTBLK=DSTBLK, Cin=Cin, Cout=Cout)
    return pl.pallas_call(
        body,
        grid=(BN // DSTBLK,),
        in_specs=[
            pl.BlockSpec((DSTBLK * KP, Cin), lambda i: (i, 0)),
            pl.BlockSpec((DSTBLK, Cin), lambda i: (i, 0)),
            pl.BlockSpec((Cin, Cout), lambda i: (0, 0)),
            pl.BlockSpec((Cin, Cout), lambda i: (0, 0)),
            pl.BlockSpec((1, Cout), lambda i: (0, 0)),
        ],
        out_specs=[
            pl.BlockSpec((DSTBLK, Cout), lambda i: (i, 0)),
            pl.BlockSpec((DSTBLK, Cout), lambda i: (i, 0)),
            pl.BlockSpec((8, Cout), lambda i: (0, 0)),
        ],
        out_shape=[
            jax.ShapeDtypeStruct((BN, Cout), jnp.float32),
            jax.ShapeDtypeStruct((BN, Cout), jnp.float32),
            jax.ShapeDtypeStruct((8, Cout), jnp.float32),
        ],
        scratch_shapes=[pltpu.VMEM((8, Cout), jnp.float32)],
    )(nbr2d, h2d, _bf(tw), _bf(pw), bias)


def _bnact_body(emax_ref, emin_ref, stats_ref, g_ref, bta_ref, out_ref):
    cnt = float(BN * K)
    mean = stats_ref[0:1, :] / cnt
    var = stats_ref[1:2, :] / cnt - mean * mean
    scale = g_ref[...] * lax.rsqrt(var + 1e-5)
    red = jnp.where(scale >= 0, emax_ref[...], emin_ref[...])
    out_ref[...] = _leaky(scale * (red - mean) + bta_ref[...])


def _bnact(emax, emin, stats, g, bta, Cout):
    RB = 1024
    return pl.pallas_call(
        _bnact_body,
        grid=(BN // RB,),
        in_specs=[
            pl.BlockSpec((RB, Cout), lambda i: (i, 0)),
            pl.BlockSpec((RB, Cout), lambda i: (i, 0)),
            pl.BlockSpec((8, Cout), lambda i: (0, 0)),
            pl.BlockSpec((1, Cout), lambda i: (0, 0)),
            pl.BlockSpec((1, Cout), lambda i: (0, 0)),
        ],
        out_specs=pl.BlockSpec((RB, Cout), lambda i: (i, 0)),
        out_shape=jax.ShapeDtypeStruct((BN, Cout), jnp.float32),
    )(emax, emin, stats, g.reshape(1, -1), bta.reshape(1, -1))


def _proj_body(h0_ref, h1_ref, h2_ref, h3_ref, w_ref, b_ref, out_ref):
    hcat = jnp.concatenate([h0_ref[...], h1_ref[...], h2_ref[...],
                            h3_ref[...]], axis=1)
    p = (jnp.dot(_bf(hcat), w_ref[...], preferred_element_type=jnp.float32)
         + b_ref[...])
    mx = jnp.max(p, axis=0, keepdims=True)
    mean = jnp.sum(p, axis=0, keepdims=True) / float(N)
    out_ref[0, 0:1, :] = mx
    out_ref[0, 1:2, :] = mean


def _proj_pool(hs, w, b):
    E = w.shape[1]
    out = pl.pallas_call(
        _proj_body,
        grid=(B,),
        in_specs=[pl.BlockSpec((1024, c), lambda i: (i, 0))
                  for c in FEATURE_DIMS]
        + [pl.BlockSpec((sum(FEATURE_DIMS), E), lambda i: (0, 0)),
           pl.BlockSpec((1, E), lambda i: (0, 0))],
        out_specs=pl.BlockSpec((1, 2, E), lambda i: (i, 0, 0)),
        out_shape=jax.ShapeDtypeStruct((B, 2, E), jnp.float32),
    )(*hs, _bf(w), b.reshape(1, -1))
    return jnp.concatenate([out[:, 0, :], out[:, 1, :]], axis=1)


def _head_body(h_ref, w0_ref, b0_ref, w1_ref, b1_ref, w2_ref, b2_ref, o_ref):
    h = h_ref[...]
    h = _leaky(jnp.dot(_bf(h), w0_ref[...],
                       preferred_element_type=jnp.float32) + b0_ref[...])
    h = _leaky(jnp.dot(_bf(h), w1_ref[...],
                       preferred_element_type=jnp.float32) + b1_ref[...])
    o_ref[...] = (jnp.dot(_bf(h), w2_ref[...],
                          preferred_element_type=jnp.float32) + b2_ref[...])


def _head(h2, w0, b0, w1, b1, w2, b2):
    return pl.pallas_call(
        _head_body,
        out_shape=jax.ShapeDtypeStruct((B, w2.shape[1]), jnp.float32),
    )(h2, _bf(w0), b0.reshape(1, -1), _bf(w1), b1.reshape(1, -1),
      _bf(w2), b2.reshape(1, -1))


def _topk_idx(dist):
    NW = 32
    RG = 8
    per_w = BN // NW
    n_g = per_w // RG
    INF = jnp.float32(jnp.inf)
    mesh = plsc.VectorSubcoreMesh(core_axis_name="c", subcore_axis_name="s")

    def _scal(v, lane_i):
        return lax.squeeze(lax.slice(v, (lane_i,), (lane_i + 1,)), (0,))

    def _cnt(msk):
        return _scal(plsc.all_reduce_population_count(msk), 0)

    _DN = lax.GatherDimensionNumbers(offset_dims=(), collapsed_slice_dims=(0,),
                                     start_index_map=(0,))

    def _perm(v, idxvec):
        return lax.gather(v, idxvec[:, None], _DN, (1,),
                          mode=lax.GatherScatterMode.PROMISE_IN_BOUNDS)

    @functools.partial(
        pl.kernel, mesh=mesh,
        compiler_params=pltpu.CompilerParams(use_tc_tiling_on_sc=False, needs_layout_passes=False),
        out_type=jax.ShapeDtypeStruct((BN * KP,), jnp.int32),
        scratch_types=[
            pltpu.VMEM((RG * N,), jnp.float32),
            pltpu.VMEM((N + 16,), jnp.float32),
            pltpu.VMEM((N + 16,), jnp.int32),
            pltpu.VMEM((RG * KP + 16,), jnp.int32),
        ],
    )
    def t(dist_hbm, out_hbm, rowbuf, cand_v, cand_i, outb):
        wid = lax.axis_index("s") * 2 + lax.axis_index("c")
        lane = lax.iota(jnp.int32, 16)

        def splat_min(v):
            for sh in (8, 4, 2, 1):
                v = jnp.minimum(v, _perm(v, jnp.bitwise_xor(lane, sh)))
            return v

        def splat_max(v):
            for sh in (8, 4, 2, 1):
                v = jnp.maximum(v, _perm(v, jnp.bitwise_xor(lane, sh)))
            return v

        def row_body(r, base_row):
            rb = r * N
            row_gid = base_row + r
            m0 = rowbuf[pl.ds(rb, 16)]
            m1 = rowbuf[pl.ds(rb + 512, 16)]
            for j in range(1, 32):
                m0 = jnp.minimum(m0, rowbuf[pl.ds(rb + j * 16, 16)])
                m1 = jnp.minimum(m1, rowbuf[pl.ds(rb + 512 + j * 16, 16)])
            thv = splat_max(jnp.maximum(m0, m1))
            ptr = jnp.int32(0)
            for j in range(64):
                v = rowbuf[pl.ds(rb + j * 16, 16)]
                msk = v <= thv
                plsc.store_compressed(cand_v.at[pl.ds(ptr, 16)], v, mask=msk)
                plsc.store_compressed(cand_i.at[pl.ds(ptr, 16)],
                                      lane + (j * 16), mask=msk)
                ptr = ptr + _cnt(msk)
            cand_v[pl.ds(ptr, 16)] = jnp.full((16,), INF)
            nt = (ptr + 15) // 16

            def rev(v):
                return lax.rev(v, (0,))

            def sort16(v):
                return lax.sort(v, dimension=0)

            def merge(tc, carry):
                b0, b1 = carry
                vs = sort16(cand_v[pl.ds(tc * 16, 16)])
                lo16 = sort16(jnp.minimum(b1, rev(vs)))
                b0n = sort16(jnp.minimum(b0, rev(lo16)))
                b1n = sort16(jnp.maximum(b0, rev(lo16)))
                return b0n, b1n

            b0, b1 = lax.fori_loop(0, nt, merge,
                                   (jnp.full((16,), INF), jnp.full((16,), INF)))
            tau = _perm(b1, jnp.full((16,), 3, jnp.int32))

            def cntA(tc, m1c):
                v = cand_v[pl.ds(tc * 16, 16)]
                return m1c + _cnt(v < tau)

            m1c = lax.fori_loop(0, nt, cntA, jnp.int32(0))
            m2v = jnp.full((16,), K - m1c, jnp.int32)

            gbase = jnp.full((16,), (row_gid // N) * N, jnp.int32)
            pad = jnp.full((16,), row_gid, jnp.int32)
            outb[pl.ds(r * KP, 16)] = pad
            outb[pl.ds(r * KP + 8, 16)] = pad

            def emit(tc, carry):
                optr, eqseen = carry
                v = cand_v[pl.ds(tc * 16, 16)]
                ci = cand_i[pl.ds(tc * 16, 16)]
                lt = v < tau
                eq = v == tau
                eqc = eq.astype(jnp.int32)
                for sh in (1, 2, 4, 8):
                    shifted = _perm(eqc, jnp.maximum(lane - sh, 0))
                    eqc = eqc + jnp.where(lane >= sh, shifted, 0)
                eqc = eqc + eqseen
                em = lt | (eq & (eqc <= m2v))
                plsc.store_compressed(outb.at[pl.ds(r * KP + optr, 16)],
                                      ci + gbase, mask=em)
                return (optr + _cnt(em),
                        eqseen + plsc.all_reduce_population_count(eq))

            lax.fori_loop(0, nt, emit,
                          (jnp.int32(0), jnp.zeros((16,), jnp.int32)))
            return base_row

        def g_body(g, _):
            base_row = wid * per_w + g * RG
            pltpu.sync_copy(dist_hbm.at[pl.ds(base_row * N, RG * N)], rowbuf)
            lax.fori_loop(0, RG, row_body, base_row)
            pltpu.sync_copy(outb.at[pl.ds(0, RG * KP)],
                            out_hbm.at[pl.ds(base_row * KP, RG * KP)])
            return 0

        lax.fori_loop(0, n_g, g_body, 0)

    return t(dist.reshape(BN * N))


def _gather_nbr(h2d, idx_flat, C):
    NW = 32
    CHUNK = 128
    ROWS = BN * KP
    per_w = ROWS // NW
    n_it = per_w // CHUNK
    mesh = plsc.VectorSubcoreMesh(core_axis_name="c", subcore_axis_name="s")

    @functools.partial(
        pl.kernel, mesh=mesh,
        compiler_params=pltpu.CompilerParams(use_tc_tiling_on_sc=False, needs_layout_passes=False),
        out_type=jax.ShapeDtypeStruct((ROWS, C), jnp.float32),
        scratch_types=[
            pltpu.VMEM((CHUNK,), jnp.int32),
            pltpu.VMEM((CHUNK, C), jnp.float32),
            pltpu.SemaphoreType.DMA,
        ],
    )
    def g(idx_hbm, table_hbm, out_hbm, idx_v, rows_v, sem):
        wid = lax.axis_index("s") * 2 + lax.axis_index("c")

        def body(i, _):
            base = wid * per_w + i * CHUNK
            pltpu.sync_copy(idx_hbm.at[pl.ds(base, CHUNK)], idx_v)
            pltpu.async_copy(table_hbm.at[idx_v], rows_v, sem).wait()
            pltpu.sync_copy(rows_v, out_hbm.at[pl.ds(base, CHUNK)])
            return 0

        lax.fori_loop(0, n_it, body, 0)

    return g(idx_flat, h2d)


def _edge_conv_layer(h2d, Cin, tw, tb, pw, pb, g, bta):
    Cout = tw.shape[1]
    dist = _dist(h2d, Cin)
    idx24 = _topk_idx(dist)
    nbr = _gather_nbr(h2d, idx24, Cin)
    emax, emin, stats = _edge(nbr, h2d, tw, pw, (tb + pb).reshape(1, -1),
                              Cin, Cout)
    return _bnact(emax, emin, stats, g, bta, Cout)


def kernel(x,
           theta_W0, theta_b0, phi_W0, phi_b0, bn_g0, bn_b0,
           theta_W1, theta_b1, phi_W1, phi_b1, bn_g1, bn_b1,
           theta_W2, theta_b2, phi_W2, phi_b2, bn_g2, bn_b2,
           theta_W3, theta_b3, phi_W3, phi_b3, bn_g3, bn_b3,
           proj_W, proj_b,
           emb_W0, emb_b0, emb_W1, emb_b1,
           out_W, out_b):
    inp = dict(locals())
    h = jnp.pad(x.reshape(BN, IN_DIMS), ((0, 0), (0, 5)))
    tw0 = jnp.pad(theta_W0, ((0, 5), (0, 0)))
    pw0 = jnp.pad(phi_W0, ((0, 5), (0, 0)))
    Cin = 8
    hs = []
    for i in range(len(FEATURE_DIMS)):
        tw = tw0 if i == 0 else inp[f"theta_W{i}"]
        pw = pw0 if i == 0 else inp[f"phi_W{i}"]
        h = _edge_conv_layer(h, Cin, tw, inp[f"theta_b{i}"],
                             pw, inp[f"phi_b{i}"],
                             inp[f"bn_g{i}"], inp[f"bn_b{i}"])
        Cin = FEATURE_DIMS[i]
        hs.append(h)
    h2 = _proj_pool(hs, proj_W, proj_b)
    return _head(h2, emb_W0, emb_b0, emb_W1, emb_b1, out_W, out_b)


# speedup vs baseline: 7.9018x; 1.0358x over previous
import functools
import jax
import jax.numpy as jnp
from jax import lax
from jax.experimental import pallas as pl
from jax.experimental.pallas import tpu as pltpu
from jax.experimental.pallas import tpu_sc as plsc

K = 20
KP = 24
B, N, IN_DIMS = 16, 1024, 3
FEATURE_DIMS = [64, 64, 128, 256]
BN = B * N


def _leaky(x):
    return jnp.where(x >= 0, x, 0.2 * x)


def _bf(x):
    return x.astype(jnp.bfloat16)


def _dist_body(h_ref, dist_ref):
    h = h_ref[...]
    sq = jnp.sum(h * h, axis=1, keepdims=True)
    hb = _bf(h)
    mm = lax.dot_general(hb, hb, (((1,), (1,)), ((), ())),
                         preferred_element_type=jnp.float32)
    dist_ref[...] = (sq + sq.reshape(1, N)) - 2.0 * mm


def _dist(h2d, C):
    return pl.pallas_call(
        _dist_body,
        grid=(B,),
        in_specs=[pl.BlockSpec((N, C), lambda b: (b, 0))],
        out_specs=pl.BlockSpec((N, N), lambda b: (b, 0)),
        out_shape=jax.ShapeDtypeStruct((BN, N), jnp.float32),
    )(h2d)


def _edge_body(nbr_ref, h_ref, tw_ref, pw_ref, bias_ref,
               emax_ref, emin_ref, stats_ref, acc_ref, DSTBLK, Cin, Cout):
    i = pl.program_id(0)
    nbr = nbr_ref[...].reshape(DSTBLK, K, Cin)
    h = h_ref[...]
    diff = h[:, None, :] - nbr
    db = _bf(diff).reshape(DSTBLK * K, Cin)
    nb = _bf(nbr).reshape(DSTBLK * K, Cin)
    e = (jnp.dot(db, tw_ref[...], preferred_element_type=jnp.float32)
         + jnp.dot(nb, pw_ref[...], preferred_element_type=jnp.float32)
         + bias_ref[...])
    e = e.reshape(DSTBLK, K, Cout)
    emax_ref[...] = jnp.max(e, axis=1)
    emin_ref[...] = jnp.min(e, axis=1)
    s = jnp.sum(e.reshape(DSTBLK * K, Cout), axis=0, keepdims=True)
    s2 = jnp.sum((e * e).reshape(DSTBLK * K, Cout), axis=0, keepdims=True)

    @pl.when(i == 0)
    def _():
        acc_ref[...] = jnp.zeros_like(acc_ref)

    acc_ref[0:1, :] += s
    acc_ref[1:2, :] += s2
    stats_ref[...] = acc_ref[...]


def _edge(nbr2d, h2d, tw, pw, bias, Cin, Cout):
    DSTBLK = 128
    body = functools.partial(_edge_body, DSTBLK=DSTBLK, Cin=Cin, Cout=Cout)
    return pl.pallas_call(
        body,
        grid=(BN // DSTBLK,),
        in_specs=[
            pl.BlockSpec((DSTBLK * K, Cin), lambda i: (i, 0)),
            pl.BlockSpec((DSTBLK, Cin), lambda i: (i, 0)),
            pl.BlockSpec((Cin, Cout), lambda i: (0, 0)),
            pl.BlockSpec((Cin, Cout), lambda i: (0, 0)),
            pl.BlockSpec((1, Cout), lambda i: (0, 0)),
        ],
        out_specs=[
            pl.BlockSpec((DSTBLK, Cout), lambda i: (i, 0)),
            pl.BlockSpec((DSTBLK, Cout), lambda i: (i, 0)),
            pl.BlockSpec((8, Cout), lambda i: (0, 0)),
        ],
        out_shape=[
            jax.ShapeDtypeStruct((BN, Cout), jnp.float32),
            jax.ShapeDtypeStruct((BN, Cout), jnp.float32),
            jax.ShapeDtypeStruct((8, Cout), jnp.float32),
        ],
        scratch_shapes=[pltpu.VMEM((8, Cout), jnp.float32)],
    )(nbr2d, h2d, _bf(tw), _bf(pw), bias)


def _bnact_body(emax_ref, emin_ref, stats_ref, g_ref, bta_ref, out_ref):
    cnt = float(BN * K)
    mean = stats_ref[0:1, :] / cnt
    var = stats_ref[1:2, :] / cnt - mean * mean
    scale = g_ref[...] * lax.rsqrt(var + 1e-5)
    red = jnp.where(scale >= 0, emax_ref[...], emin_ref[...])
    out_ref[...] = _leaky(scale * (red - mean) + bta_ref[...])


def _bnact(emax, emin, stats, g, bta, Cout):
    RB = 1024
    return pl.pallas_call(
        _bnact_body,
        grid=(BN // RB,),
        in_specs=[
            pl.BlockSpec((RB, Cout), lambda i: (i, 0)),
            pl.BlockSpec((RB, Cout), lambda i: (i, 0)),
            pl.BlockSpec((8, Cout), lambda i: (0, 0)),
            pl.BlockSpec((1, Cout), lambda i: (0, 0)),
            pl.BlockSpec((1, Cout), lambda i: (0, 0)),
        ],
        out_specs=pl.BlockSpec((RB, Cout), lambda i: (i, 0)),
        out_shape=jax.ShapeDtypeStruct((BN, Cout), jnp.float32),
    )(emax, emin, stats, g.reshape(1, -1), bta.reshape(1, -1))


def _proj_body(h0_ref, h1_ref, h2_ref, h3_ref, w_ref, b_ref, out_ref):
    hcat = jnp.concatenate([h0_ref[...], h1_ref[...], h2_ref[...],
                            h3_ref[...]], axis=1)
    p = (jnp.dot(_bf(hcat), w_ref[...], preferred_element_type=jnp.float32)
         + b_ref[...])
    mx = jnp.max(p, axis=0, keepdims=True)
    mean = jnp.sum(p, axis=0, keepdims=True) / float(N)
    out_ref[0, 0:1, :] = mx
    out_ref[0, 1:2, :] = mean


def _proj_pool(hs, w, b):
    E = w.shape[1]
    out = pl.pallas_call(
        _proj_body,
        grid=(B,),
        in_specs=[pl.BlockSpec((1024, c), lambda i: (i, 0))
                  for c in FEATURE_DIMS]
        + [pl.BlockSpec((sum(FEATURE_DIMS), E), lambda i: (0, 0)),
           pl.BlockSpec((1, E), lambda i: (0, 0))],
        out_specs=pl.BlockSpec((1, 2, E), lambda i: (i, 0, 0)),
        out_shape=jax.ShapeDtypeStruct((B, 2, E), jnp.float32),
    )(*hs, _bf(w), b.reshape(1, -1))
    return jnp.concatenate([out[:, 0, :], out[:, 1, :]], axis=1)


def _head_body(h_ref, w0_ref, b0_ref, w1_ref, b1_ref, w2_ref, b2_ref, o_ref):
    h = h_ref[...]
    h = _leaky(jnp.dot(_bf(h), w0_ref[...],
                       preferred_element_type=jnp.float32) + b0_ref[...])
    h = _leaky(jnp.dot(_bf(h), w1_ref[...],
                       preferred_element_type=jnp.float32) + b1_ref[...])
    o_ref[...] = (jnp.dot(_bf(h), w2_ref[...],
                          preferred_element_type=jnp.float32) + b2_ref[...])


def _head(h2, w0, b0, w1, b1, w2, b2):
    return pl.pallas_call(
        _head_body,
        out_shape=jax.ShapeDtypeStruct((B, w2.shape[1]), jnp.float32),
    )(h2, _bf(w0), b0.reshape(1, -1), _bf(w1), b1.reshape(1, -1),
      _bf(w2), b2.reshape(1, -1))


def _topk_idx(dist):
    NW = 32
    RG = 8
    per_w = BN // NW
    n_g = per_w // RG
    INF = jnp.float32(jnp.inf)
    mesh = plsc.VectorSubcoreMesh(core_axis_name="c", subcore_axis_name="s")

    def _scal(v, lane_i):
        return lax.squeeze(lax.slice(v, (lane_i,), (lane_i + 1,)), (0,))

    def _cnt(msk):
        return _scal(plsc.all_reduce_population_count(msk), 0)

    _DN = lax.GatherDimensionNumbers(offset_dims=(), collapsed_slice_dims=(0,),
                                     start_index_map=(0,))

    def _perm(v, idxvec):
        return lax.gather(v, idxvec[:, None], _DN, (1,),
                          mode=lax.GatherScatterMode.PROMISE_IN_BOUNDS)

    @functools.partial(
        pl.kernel, mesh=mesh,
        compiler_params=pltpu.CompilerParams(use_tc_tiling_on_sc=False, needs_layout_passes=False),
        out_type=jax.ShapeDtypeStruct((BN * K,), jnp.int32),
        scratch_types=[
            pltpu.VMEM((RG * N,), jnp.float32),
            pltpu.VMEM((N + 16,), jnp.float32),
            pltpu.VMEM((N + 16,), jnp.int32),
            pltpu.VMEM((RG * K + 16,), jnp.int32),
        ],
    )
    def t(dist_hbm, out_hbm, rowbuf, cand_v, cand_i, outb):
        wid = lax.axis_index("s") * 2 + lax.axis_index("c")
        lane = lax.iota(jnp.int32, 16)

        def splat_min(v):
            for sh in (8, 4, 2, 1):
                v = jnp.minimum(v, _perm(v, jnp.bitwise_xor(lane, sh)))
            return v

        def splat_max(v):
            for sh in (8, 4, 2, 1):
                v = jnp.maximum(v, _perm(v, jnp.bitwise_xor(lane, sh)))
            return v

        def row_body(r, base_row):
            rb = r * N
            row_gid = base_row + r
            m0 = rowbuf[pl.ds(rb, 16)]
            m1 = rowbuf[pl.ds(rb + 512, 16)]
            for j in range(1, 32):
                m0 = jnp.minimum(m0, rowbuf[pl.ds(rb + j * 16, 16)])
                m1 = jnp.minimum(m1, rowbuf[pl.ds(rb + 512 + j * 16, 16)])
            thv = splat_max(jnp.maximum(m0, m1))
            ptr = jnp.int32(0)
            for j in range(64):
                v = rowbuf[pl.ds(rb + j * 16, 16)]
                msk = v <= thv
                plsc.store_compressed(cand_v.at[pl.ds(ptr, 16)], v, mask=msk)
                plsc.store_compressed(cand_i.at[pl.ds(ptr, 16)],
                                      lane + (j * 16), mask=msk)
                ptr = ptr + _cnt(msk)
            cand_v[pl.ds(ptr, 16)] = jnp.full((16,), INF)
            nt = (ptr + 15) // 16

            def rev(v):
                return lax.rev(v, (0,))

            def sort16(v):
                return lax.sort(v, dimension=0)

            def merge(tc, carry):
                b0, b1 = carry
                vs = sort16(cand_v[pl.ds(tc * 16, 16)])
                lo16 = sort16(jnp.minimum(b1, rev(vs)))
                b0n = sort16(jnp.minimum(b0, rev(lo16)))
                b1n = sort16(jnp.maximum(b0, rev(lo16)))
                return b0n, b1n

            b0, b1 = lax.fori_loop(0, nt, merge,
                                   (jnp.full((16,), INF), jnp.full((16,), INF)))
            tau = _perm(b1, jnp.full((16,), 3, jnp.int32))

            def cntA(tc, m1c):
                v = cand_v[pl.ds(tc * 16, 16)]
                return m1c + _cnt(v < tau)

            m1c = lax.fori_loop(0, nt, cntA, jnp.int32(0))
            m2v = jnp.full((16,), K - m1c, jnp.int32)

            gbase = jnp.full((16,), (row_gid // N) * N, jnp.int32)

            def emit(tc, carry):
                optr, eqseen = carry
                v = cand_v[pl.ds(tc * 16, 16)]
                ci = cand_i[pl.ds(tc * 16, 16)]
                lt = v < tau
                eq = v == tau
                eqc = eq.astype(jnp.int32)
                for sh in (1, 2, 4, 8):
                    shifted = _perm(eqc, jnp.maximum(lane - sh, 0))
                    eqc = eqc + jnp.where(lane >= sh, shifted, 0)
                eqc = eqc + eqseen
                em = lt | (eq & (eqc <= m2v))
                plsc.store_compressed(outb.at[pl.ds(r * K + optr, 16)],
                                      ci + gbase, mask=em)
                return (optr + _cnt(em),
                        eqseen + plsc.all_reduce_population_count(eq))

            lax.fori_loop(0, nt, emit,
                          (jnp.int32(0), jnp.zeros((16,), jnp.int32)))
            return base_row

        def g_body(g, _):
            base_row = wid * per_w + g * RG
            pltpu.sync_copy(dist_hbm.at[pl.ds(base_row * N, RG * N)], rowbuf)
            lax.fori_loop(0, RG, row_body, base_row)
            pltpu.sync_copy(outb.at[pl.ds(0, RG * K)],
                            out_hbm.at[pl.ds(base_row * K, RG * K)])
            return 0

        lax.fori_loop(0, n_g, g_body, 0)

    return t(dist.reshape(BN * N))


def _gather_nbr(h2d, idx_flat, C):
    NW = 32
    CHUNK = 128
    ROWS = BN * K
    per_w = ROWS // NW
    n_it = per_w // CHUNK
    mesh = plsc.VectorSubcoreMesh(core_axis_name="c", subcore_axis_name="s")

    @functools.partial(
        pl.kernel, mesh=mesh,
        compiler_params=pltpu.CompilerParams(use_tc_tiling_on_sc=False, needs_layout_passes=False),
        out_type=jax.ShapeDtypeStruct((ROWS, C), jnp.float32),
        scratch_types=[
            pltpu.VMEM((CHUNK,), jnp.int32),
            pltpu.VMEM((CHUNK, C), jnp.float32),
            pltpu.SemaphoreType.DMA,
        ],
    )
    def g(idx_hbm, table_hbm, out_hbm, idx_v, rows_v, sem):
        wid = lax.axis_index("s") * 2 + lax.axis_index("c")

        def body(i, _):
            base = wid * per_w + i * CHUNK
            pltpu.sync_copy(idx_hbm.at[pl.ds(base, CHUNK)], idx_v)
            pltpu.async_copy(table_hbm.at[idx_v], rows_v, sem).wait()
            pltpu.sync_copy(rows_v, out_hbm.at[pl.ds(base, CHUNK)])
            return 0

        lax.fori_loop(0, n_it, body, 0)

    return g(idx_flat, h2d)


def _edge_conv_layer(h2d, Cin, tw, tb, pw, pb, g, bta):
    Cout = tw.shape[1]
    dist = _dist(h2d, Cin)
    idx24 = _topk_idx(dist)
    nbr = _gather_nbr(h2d, idx24, Cin)
    emax, emin, stats = _edge(nbr, h2d, tw, pw, (tb + pb).reshape(1, -1),
                              Cin, Cout)
    return _bnact(emax, emin, stats, g, bta, Cout)


def kernel(x,
           theta_W0, theta_b0, phi_W0, phi_b0, bn_g0, bn_b0,
           theta_W1, theta_b1, phi_W1, phi_b1, bn_g1, bn_b1,
           theta_W2, theta_b2, phi_W2, phi_b2, bn_g2, bn_b2,
           theta_W3, theta_b3, phi_W3, phi_b3, bn_g3, bn_b3,
           proj_W, proj_b,
           emb_W0, emb_b0, emb_W1, emb_b1,
           out_W, out_b):
    inp = dict(locals())
    h = jnp.pad(x.reshape(BN, IN_DIMS), ((0, 0), (0, 5)))
    tw0 = jnp.pad(theta_W0, ((0, 5), (0, 0)))
    pw0 = jnp.pad(phi_W0, ((0, 5), (0, 0)))
    Cin = 8
    hs = []
    for i in range(len(FEATURE_DIMS)):
        tw = tw0 if i == 0 else inp[f"theta_W{i}"]
        pw = pw0 if i == 0 else inp[f"phi_W{i}"]
        h = _edge_conv_layer(h, Cin, tw, inp[f"theta_b{i}"],
                             pw, inp[f"phi_b{i}"],
                             inp[f"bn_g{i}"], inp[f"bn_b{i}"])
        Cin = FEATURE_DIMS[i]
        hs.append(h)
    h2 = _proj_pool(hs, proj_W, proj_b)
    return _head(h2, emb_W0, emb_b0, emb_W1, emb_b1, out_W, out_b)


# speedup vs baseline: 8.0724x; 1.0216x over previous
import functools
import jax
import jax.numpy as jnp
from jax import lax
from jax.experimental import pallas as pl
from jax.experimental.pallas import tpu as pltpu
from jax.experimental.pallas import tpu_sc as plsc

K = 20
KP = 24
B, N, IN_DIMS = 16, 1024, 3
FEATURE_DIMS = [64, 64, 128, 256]
BN = B * N


def _leaky(x):
    return jnp.where(x >= 0, x, 0.2 * x)


def _bf(x):
    return x.astype(jnp.bfloat16)


def _dist_body(h_ref, dist_ref):
    h = h_ref[...]
    sq = jnp.sum(h * h, axis=1, keepdims=True)
    hb = _bf(h)
    mm = lax.dot_general(hb, hb, (((1,), (1,)), ((), ())),
                         preferred_element_type=jnp.float32)
    dist_ref[...] = (sq + sq.reshape(1, N)) - 2.0 * mm


def _dist(h2d, C):
    return pl.pallas_call(
        _dist_body,
        grid=(B,),
        in_specs=[pl.BlockSpec((N, C), lambda b: (b, 0))],
        out_specs=pl.BlockSpec((N, N), lambda b: (b, 0)),
        out_shape=jax.ShapeDtypeStruct((BN, N), jnp.float32),
    )(h2d)


def _edge_body(nbr_ref, h_ref, tw_ref, pw_ref, bias_ref,
               emax_ref, emin_ref, stats_ref, acc_ref, DSTBLK, Cin, Cout):
    i = pl.program_id(0)
    nbr = nbr_ref[...].reshape(DSTBLK, K, Cin)
    h = h_ref[...]
    diff = h[:, None, :] - nbr
    db = _bf(diff).reshape(DSTBLK * K, Cin)
    nb = _bf(nbr).reshape(DSTBLK * K, Cin)
    e = (jnp.dot(db, tw_ref[...], preferred_element_type=jnp.float32)
         + jnp.dot(nb, pw_ref[...], preferred_element_type=jnp.float32)
         + bias_ref[...])
    e = e.reshape(DSTBLK, K, Cout)
    emax_ref[...] = jnp.max(e, axis=1)
    emin_ref[...] = jnp.min(e, axis=1)
    s = jnp.sum(e.reshape(DSTBLK * K, Cout), axis=0, keepdims=True)
    s2 = jnp.sum((e * e).reshape(DSTBLK * K, Cout), axis=0, keepdims=True)

    @pl.when(i == 0)
    def _():
        acc_ref[...] = jnp.zeros_like(acc_ref)

    acc_ref[0:1, :] += s
    acc_ref[1:2, :] += s2
    stats_ref[...] = acc_ref[...]


def _edge(nbr2d, h2d, tw, pw, bias, Cin, Cout):
    DSTBLK = 256
    body = functools.partial(_edge_body, DSTBLK=DSTBLK, Cin=Cin, Cout=Cout)
    return pl.pallas_call(
        body,
        grid=(BN // DSTBLK,),
        in_specs=[
            pl.BlockSpec((DSTBLK * K, Cin), lambda i: (i, 0)),
            pl.BlockSpec((DSTBLK, Cin), lambda i: (i, 0)),
            pl.BlockSpec((Cin, Cout), lambda i: (0, 0)),
            pl.BlockSpec((Cin, Cout), lambda i: (0, 0)),
            pl.BlockSpec((1, Cout), lambda i: (0, 0)),
        ],
        out_specs=[
            pl.BlockSpec((DSTBLK, Cout), lambda i: (i, 0)),
            pl.BlockSpec((DSTBLK, Cout), lambda i: (i, 0)),
            pl.BlockSpec((8, Cout), lambda i: (0, 0)),
        ],
        out_shape=[
            jax.ShapeDtypeStruct((BN, Cout), jnp.float32),
            jax.ShapeDtypeStruct((BN, Cout), jnp.float32),
            jax.ShapeDtypeStruct((8, Cout), jnp.float32),
        ],
        scratch_shapes=[pltpu.VMEM((8, Cout), jnp.float32)],
    )(nbr2d, h2d, _bf(tw), _bf(pw), bias)


def _bnact_body(emax_ref, emin_ref, stats_ref, g_ref, bta_ref, out_ref):
    cnt = float(BN * K)
    mean = stats_ref[0:1, :] / cnt
    var = stats_ref[1:2, :] / cnt - mean * mean
    scale = g_ref[...] * lax.rsqrt(var + 1e-5)
    red = jnp.where(scale >= 0, emax_ref[...], emin_ref[...])
    out_ref[...] = _leaky(scale * (red - mean) + bta_ref[...])


def _bnact(emax, emin, stats, g, bta, Cout):
    RB = 1024
    return pl.pallas_call(
        _bnact_body,
        grid=(BN // RB,),
        in_specs=[
            pl.BlockSpec((RB, Cout), lambda i: (i, 0)),
            pl.BlockSpec((RB, Cout), lambda i: (i, 0)),
            pl.BlockSpec((8, Cout), lambda i: (0, 0)),
            pl.BlockSpec((1, Cout), lambda i: (0, 0)),
            pl.BlockSpec((1, Cout), lambda i: (0, 0)),
        ],
        out_specs=pl.BlockSpec((RB, Cout), lambda i: (i, 0)),
        out_shape=jax.ShapeDtypeStruct((BN, Cout), jnp.float32),
    )(emax, emin, stats, g.reshape(1, -1), bta.reshape(1, -1))


def _proj_body(h0_ref, h1_ref, h2_ref, h3_ref, w_ref, b_ref, out_ref):
    hcat = jnp.concatenate([h0_ref[...], h1_ref[...], h2_ref[...],
                            h3_ref[...]], axis=1)
    p = (jnp.dot(_bf(hcat), w_ref[...], preferred_element_type=jnp.float32)
         + b_ref[...])
    mx = jnp.max(p, axis=0, keepdims=True)
    mean = jnp.sum(p, axis=0, keepdims=True) / float(N)
    out_ref[0, 0:1, :] = mx
    out_ref[0, 1:2, :] = mean


def _proj_pool(hs, w, b):
    E = w.shape[1]
    out = pl.pallas_call(
        _proj_body,
        grid=(B,),
        in_specs=[pl.BlockSpec((1024, c), lambda i: (i, 0))
                  for c in FEATURE_DIMS]
        + [pl.BlockSpec((sum(FEATURE_DIMS), E), lambda i: (0, 0)),
           pl.BlockSpec((1, E), lambda i: (0, 0))],
        out_specs=pl.BlockSpec((1, 2, E), lambda i: (i, 0, 0)),
        out_shape=jax.ShapeDtypeStruct((B, 2, E), jnp.float32),
    )(*hs, _bf(w), b.reshape(1, -1))
    return jnp.concatenate([out[:, 0, :], out[:, 1, :]], axis=1)


def _head_body(h_ref, w0_ref, b0_ref, w1_ref, b1_ref, w2_ref, b2_ref, o_ref):
    h = h_ref[...]
    h = _leaky(jnp.dot(_bf(h), w0_ref[...],
                       preferred_element_type=jnp.float32) + b0_ref[...])
    h = _leaky(jnp.dot(_bf(h), w1_ref[...],
                       preferred_element_type=jnp.float32) + b1_ref[...])
    o_ref[...] = (jnp.dot(_bf(h), w2_ref[...],
                          preferred_element_type=jnp.float32) + b2_ref[...])


def _head(h2, w0, b0, w1, b1, w2, b2):
    return pl.pallas_call(
        _head_body,
        out_shape=jax.ShapeDtypeStruct((B, w2.shape[1]), jnp.float32),
    )(h2, _bf(w0), b0.reshape(1, -1), _bf(w1), b1.reshape(1, -1),
      _bf(w2), b2.reshape(1, -1))


def _topk_idx(dist):
    NW = 32
    RG = 16
    per_w = BN // NW
    n_g = per_w // RG
    INF = jnp.float32(jnp.inf)
    mesh = plsc.VectorSubcoreMesh(core_axis_name="c", subcore_axis_name="s")

    def _scal(v, lane_i):
        return lax.squeeze(lax.slice(v, (lane_i,), (lane_i + 1,)), (0,))

    def _cnt(msk):
        return _scal(plsc.all_reduce_population_count(msk), 0)

    _DN = lax.GatherDimensionNumbers(offset_dims=(), collapsed_slice_dims=(0,),
                                     start_index_map=(0,))

    def _perm(v, idxvec):
        return lax.gather(v, idxvec[:, None], _DN, (1,),
                          mode=lax.GatherScatterMode.PROMISE_IN_BOUNDS)

    @functools.partial(
        pl.kernel, mesh=mesh,
        compiler_params=pltpu.CompilerParams(use_tc_tiling_on_sc=False, needs_layout_passes=False),
        out_type=jax.ShapeDtypeStruct((BN * K,), jnp.int32),
        scratch_types=[
            pltpu.VMEM((RG * N,), jnp.float32),
            pltpu.VMEM((N + 16,), jnp.float32),
            pltpu.VMEM((N + 16,), jnp.int32),
            pltpu.VMEM((RG * K + 16,), jnp.int32),
        ],
    )
    def t(dist_hbm, out_hbm, rowbuf, cand_v, cand_i, outb):
        wid = lax.axis_index("s") * 2 + lax.axis_index("c")
        lane = lax.iota(jnp.int32, 16)

        def splat_min(v):
            for sh in (8, 4, 2, 1):
                v = jnp.minimum(v, _perm(v, jnp.bitwise_xor(lane, sh)))
            return v

        def splat_max(v):
            for sh in (8, 4, 2, 1):
                v = jnp.maximum(v, _perm(v, jnp.bitwise_xor(lane, sh)))
            return v

        def row_body(r, base_row):
            rb = r * N
            row_gid = base_row + r
            m0 = rowbuf[pl.ds(rb, 16)]
            m1 = rowbuf[pl.ds(rb + 512, 16)]
            for j in range(1, 32):
                m0 = jnp.minimum(m0, rowbuf[pl.ds(rb + j * 16, 16)])
                m1 = jnp.minimum(m1, rowbuf[pl.ds(rb + 512 + j * 16, 16)])
            thv = splat_max(jnp.maximum(m0, m1))
            ptr = jnp.int32(0)
            for j in range(64):
                v = rowbuf[pl.ds(rb + j * 16, 16)]
                msk = v <= thv
                plsc.store_compressed(cand_v.at[pl.ds(ptr, 16)], v, mask=msk)
                plsc.store_compressed(cand_i.at[pl.ds(ptr, 16)],
                                      lane + (j * 16), mask=msk)
                ptr = ptr + _cnt(msk)
            cand_v[pl.ds(ptr, 16)] = jnp.full((16,), INF)
            nt = (ptr + 15) // 16

            def rev(v):
                return lax.rev(v, (0,))

            def sort16(v):
                return lax.sort(v, dimension=0)

            def merge(tc, carry):
                b0, b1 = carry
                vs = sort16(cand_v[pl.ds(tc * 16, 16)])
                lo16 = sort16(jnp.minimum(b1, rev(vs)))
                b0n = sort16(jnp.minimum(b0, rev(lo16)))
                b1n = sort16(jnp.maximum(b0, rev(lo16)))
                return b0n, b1n

            b0, b1 = lax.fori_loop(0, nt, merge,
                                   (jnp.full((16,), INF), jnp.full((16,), INF)))
            tau = _perm(b1, jnp.full((16,), 3, jnp.int32))

            def cntA(tc, m1c):
                v = cand_v[pl.ds(tc * 16, 16)]
                return m1c + _cnt(v < tau)

            m1c = lax.fori_loop(0, nt, cntA, jnp.int32(0))
            m2v = jnp.full((16,), K - m1c, jnp.int32)

            gbase = jnp.full((16,), (row_gid // N) * N, jnp.int32)

            def emit(tc, carry):
                optr, eqseen = carry
                v = cand_v[pl.ds(tc * 16, 16)]
                ci = cand_i[pl.ds(tc * 16, 16)]
                lt = v < tau
                eq = v == tau
                eqc = eq.astype(jnp.int32)
                for sh in (1, 2, 4, 8):
                    shifted = _perm(eqc, jnp.maximum(lane - sh, 0))
                    eqc = eqc + jnp.where(lane >= sh, shifted, 0)
                eqc = eqc + eqseen
                em = lt | (eq & (eqc <= m2v))
                plsc.store_compressed(outb.at[pl.ds(r * K + optr, 16)],
                                      ci + gbase, mask=em)
                return (optr + _cnt(em),
                        eqseen + plsc.all_reduce_population_count(eq))

            lax.fori_loop(0, nt, emit,
                          (jnp.int32(0), jnp.zeros((16,), jnp.int32)))
            return base_row

        def g_body(g, _):
            base_row = wid * per_w + g * RG
            pltpu.sync_copy(dist_hbm.at[pl.ds(base_row * N, RG * N)], rowbuf)
            lax.fori_loop(0, RG, row_body, base_row)
            pltpu.sync_copy(outb.at[pl.ds(0, RG * K)],
                            out_hbm.at[pl.ds(base_row * K, RG * K)])
            return 0

        lax.fori_loop(0, n_g, g_body, 0)

    return t(dist.reshape(BN * N))


def _gather_nbr(h2d, idx_flat, C):
    NW = 32
    CHUNK = 128
    ROWS = BN * K
    per_w = ROWS // NW
    n_it = per_w // CHUNK
    mesh = plsc.VectorSubcoreMesh(core_axis_name="c", subcore_axis_name="s")

    @functools.partial(
        pl.kernel, mesh=mesh,
        compiler_params=pltpu.CompilerParams(use_tc_tiling_on_sc=False, needs_layout_passes=False),
        out_type=jax.ShapeDtypeStruct((ROWS, C), jnp.float32),
        scratch_types=[
            pltpu.VMEM((CHUNK,), jnp.int32),
            pltpu.VMEM((CHUNK, C), jnp.float32),
            pltpu.SemaphoreType.DMA,
        ],
    )
    def g(idx_hbm, table_hbm, out_hbm, idx_v, rows_v, sem):
        wid = lax.axis_index("s") * 2 + lax.axis_index("c")

        def body(i, _):
            base = wid * per_w + i * CHUNK
            pltpu.sync_copy(idx_hbm.at[pl.ds(base, CHUNK)], idx_v)
            pltpu.async_copy(table_hbm.at[idx_v], rows_v, sem).wait()
            pltpu.sync_copy(rows_v, out_hbm.at[pl.ds(base, CHUNK)])
            return 0

        lax.fori_loop(0, n_it, body, 0)

    return g(idx_flat, h2d)


def _edge_conv_layer(h2d, Cin, tw, tb, pw, pb, g, bta):
    Cout = tw.shape[1]
    dist = _dist(h2d, Cin)
    idx24 = _topk_idx(dist)
    nbr = _gather_nbr(h2d, idx24, Cin)
    emax, emin, stats = _edge(nbr, h2d, tw, pw, (tb + pb).reshape(1, -1),
                              Cin, Cout)
    return _bnact(emax, emin, stats, g, bta, Cout)


def kernel(x,
           theta_W0, theta_b0, phi_W0, phi_b0, bn_g0, bn_b0,
           theta_W1, theta_b1, phi_W1, phi_b1, bn_g1, bn_b1,
           theta_W2, theta_b2, phi_W2, phi_b2, bn_g2, bn_b2,
           theta_W3, theta_b3, phi_W3, phi_b3, bn_g3, bn_b3,
           proj_W, proj_b,
           emb_W0, emb_b0, emb_W1, emb_b1,
           out_W, out_b):
    inp = dict(locals())
    h = jnp.pad(x.reshape(BN, IN_DIMS), ((0, 0), (0, 5)))
    tw0 = jnp.pad(theta_W0, ((0, 5), (0, 0)))
    pw0 = jnp.pad(phi_W0, ((0, 5), (0, 0)))
    Cin = 8
    hs = []
    for i in range(len(FEATURE_DIMS)):
        tw = tw0 if i == 0 else inp[f"theta_W{i}"]
        pw = pw0 if i == 0 else inp[f"phi_W{i}"]
        h = _edge_conv_layer(h, Cin, tw, inp[f"theta_b{i}"],
                             pw, inp[f"phi_b{i}"],
                             inp[f"bn_g{i}"], inp[f"bn_b{i}"])
        Cin = FEATURE_DIMS[i]
        hs.append(h)
    h2 = _proj_pool(hs, proj_W, proj_b)
    return _head(h2, emb_W0, emb_b0, emb_W1, emb_b1, out_W, out_b)


# speedup vs baseline: 8.5469x; 1.0588x over previous
import functools
import jax
import jax.numpy as jnp
from jax import lax
from jax.experimental import pallas as pl
from jax.experimental.pallas import tpu as pltpu
from jax.experimental.pallas import tpu_sc as plsc

K = 20
KP = 24
B, N, IN_DIMS = 16, 1024, 3
FEATURE_DIMS = [64, 64, 128, 256]
BN = B * N


def _leaky(x):
    return jnp.where(x >= 0, x, 0.2 * x)


def _bf(x):
    return x.astype(jnp.bfloat16)


def _dist_body(h_ref, dist_ref):
    h = h_ref[...]
    sq = jnp.sum(h * h, axis=1, keepdims=True)
    hb = _bf(h)
    mm = lax.dot_general(hb, hb, (((1,), (1,)), ((), ())),
                         preferred_element_type=jnp.float32)
    dist_ref[...] = (sq + sq.reshape(1, N)) - 2.0 * mm


def _dist(h2d, C):
    return pl.pallas_call(
        _dist_body,
        grid=(B,),
        in_specs=[pl.BlockSpec((N, C), lambda b: (b, 0))],
        out_specs=pl.BlockSpec((N, N), lambda b: (b, 0)),
        out_shape=jax.ShapeDtypeStruct((BN, N), jnp.float32),
    )(h2d)


def _edge_body(nbr_ref, h_ref, tw_ref, pw_ref, bias_ref,
               emax_ref, emin_ref, stats_ref, acc_ref, DSTBLK, Cin, Cout):
    i = pl.program_id(0)
    nbr = nbr_ref[...].reshape(DSTBLK, K, Cin)
    h = h_ref[...]
    diff = h[:, None, :] - nbr
    db = _bf(diff).reshape(DSTBLK * K, Cin)
    nb = _bf(nbr).reshape(DSTBLK * K, Cin)
    e = (jnp.dot(db, tw_ref[...], preferred_element_type=jnp.float32)
         + jnp.dot(nb, pw_ref[...], preferred_element_type=jnp.float32)
         + bias_ref[...])
    e = e.reshape(DSTBLK, K, Cout)
    emax_ref[...] = jnp.max(e, axis=1)
    emin_ref[...] = jnp.min(e, axis=1)
    s = jnp.sum(e.reshape(DSTBLK * K, Cout), axis=0, keepdims=True)
    s2 = jnp.sum((e * e).reshape(DSTBLK * K, Cout), axis=0, keepdims=True)

    @pl.when(i == 0)
    def _():
        acc_ref[...] = jnp.zeros_like(acc_ref)

    acc_ref[0:1, :] += s
    acc_ref[1:2, :] += s2
    stats_ref[...] = acc_ref[...]


def _edge(nbr2d, h2d, tw, pw, bias, Cin, Cout):
    DSTBLK = 256
    body = functools.partial(_edge_body, DSTBLK=DSTBLK, Cin=Cin, Cout=Cout)
    return pl.pallas_call(
        body,
        grid=(BN // DSTBLK,),
        in_specs=[
            pl.BlockSpec((DSTBLK * K, Cin), lambda i: (i, 0)),
            pl.BlockSpec((DSTBLK, Cin), lambda i: (i, 0)),
            pl.BlockSpec((Cin, Cout), lambda i: (0, 0)),
            pl.BlockSpec((Cin, Cout), lambda i: (0, 0)),
            pl.BlockSpec((1, Cout), lambda i: (0, 0)),
        ],
        out_specs=[
            pl.BlockSpec((DSTBLK, Cout), lambda i: (i, 0)),
            pl.BlockSpec((DSTBLK, Cout), lambda i: (i, 0)),
            pl.BlockSpec((8, Cout), lambda i: (0, 0)),
        ],
        out_shape=[
            jax.ShapeDtypeStruct((BN, Cout), jnp.float32),
            jax.ShapeDtypeStruct((BN, Cout), jnp.float32),
            jax.ShapeDtypeStruct((8, Cout), jnp.float32),
        ],
        scratch_shapes=[pltpu.VMEM((8, Cout), jnp.float32)],
    )(nbr2d, h2d, _bf(tw), _bf(pw), bias)


def _bnact_body(emax_ref, emin_ref, stats_ref, g_ref, bta_ref, out_ref):
    cnt = float(BN * K)
    mean = stats_ref[0:1, :] / cnt
    var = stats_ref[1:2, :] / cnt - mean * mean
    scale = g_ref[...] * lax.rsqrt(var + 1e-5)
    red = jnp.where(scale >= 0, emax_ref[...], emin_ref[...])
    out_ref[...] = _leaky(scale * (red - mean) + bta_ref[...])


def _bnact(emax, emin, stats, g, bta, Cout):
    RB = 1024
    return pl.pallas_call(
        _bnact_body,
        grid=(BN // RB,),
        in_specs=[
            pl.BlockSpec((RB, Cout), lambda i: (i, 0)),
            pl.BlockSpec((RB, Cout), lambda i: (i, 0)),
            pl.BlockSpec((8, Cout), lambda i: (0, 0)),
            pl.BlockSpec((1, Cout), lambda i: (0, 0)),
            pl.BlockSpec((1, Cout), lambda i: (0, 0)),
        ],
        out_specs=pl.BlockSpec((RB, Cout), lambda i: (i, 0)),
        out_shape=jax.ShapeDtypeStruct((BN, Cout), jnp.float32),
    )(emax, emin, stats, g.reshape(1, -1), bta.reshape(1, -1))


def _proj_body(h0_ref, h1_ref, h2_ref, h3_ref, w_ref, b_ref, out_ref):
    hcat = jnp.concatenate([h0_ref[...], h1_ref[...], h2_ref[...],
                            h3_ref[...]], axis=1)
    p = (jnp.dot(_bf(hcat), w_ref[...], preferred_element_type=jnp.float32)
         + b_ref[...])
    mx = jnp.max(p, axis=0, keepdims=True)
    mean = jnp.sum(p, axis=0, keepdims=True) / float(N)
    out_ref[0, 0:1, :] = mx
    out_ref[0, 1:2, :] = mean


def _proj_pool(hs, w, b):
    E = w.shape[1]
    out = pl.pallas_call(
        _proj_body,
        grid=(B,),
        in_specs=[pl.BlockSpec((1024, c), lambda i: (i, 0))
                  for c in FEATURE_DIMS]
        + [pl.BlockSpec((sum(FEATURE_DIMS), E), lambda i: (0, 0)),
           pl.BlockSpec((1, E), lambda i: (0, 0))],
        out_specs=pl.BlockSpec((1, 2, E), lambda i: (i, 0, 0)),
        out_shape=jax.ShapeDtypeStruct((B, 2, E), jnp.float32),
    )(*hs, _bf(w), b.reshape(1, -1))
    return jnp.concatenate([out[:, 0, :], out[:, 1, :]], axis=1)


def _head_body(h_ref, w0_ref, b0_ref, w1_ref, b1_ref, w2_ref, b2_ref, o_ref):
    h = h_ref[...]
    h = _leaky(jnp.dot(_bf(h), w0_ref[...],
                       preferred_element_type=jnp.float32) + b0_ref[...])
    h = _leaky(jnp.dot(_bf(h), w1_ref[...],
                       preferred_element_type=jnp.float32) + b1_ref[...])
    o_ref[...] = (jnp.dot(_bf(h), w2_ref[...],
                          preferred_element_type=jnp.float32) + b2_ref[...])


def _head(h2, w0, b0, w1, b1, w2, b2):
    return pl.pallas_call(
        _head_body,
        out_shape=jax.ShapeDtypeStruct((B, w2.shape[1]), jnp.float32),
    )(h2, _bf(w0), b0.reshape(1, -1), _bf(w1), b1.reshape(1, -1),
      _bf(w2), b2.reshape(1, -1))


def _topk_idx(dist):
    NW = 32
    RG = 16
    per_w = BN // NW
    n_g = per_w // RG
    INF = jnp.float32(jnp.inf)
    mesh = plsc.VectorSubcoreMesh(core_axis_name="c", subcore_axis_name="s")

    def _scal(v, lane_i):
        return lax.squeeze(lax.slice(v, (lane_i,), (lane_i + 1,)), (0,))

    def _cnt(msk):
        return _scal(plsc.all_reduce_population_count(msk), 0)

    _DN = lax.GatherDimensionNumbers(offset_dims=(), collapsed_slice_dims=(0,),
                                     start_index_map=(0,))

    def _perm(v, idxvec):
        return lax.gather(v, idxvec[:, None], _DN, (1,),
                          mode=lax.GatherScatterMode.PROMISE_IN_BOUNDS)

    @functools.partial(
        pl.kernel, mesh=mesh,
        compiler_params=pltpu.CompilerParams(use_tc_tiling_on_sc=False, needs_layout_passes=False),
        out_type=jax.ShapeDtypeStruct((BN * K,), jnp.int32),
        scratch_types=[
            pltpu.VMEM((RG * N,), jnp.float32),
            pltpu.VMEM((N + 16,), jnp.float32),
            pltpu.VMEM((N + 16,), jnp.int32),
            pltpu.VMEM((RG * K + 16,), jnp.int32),
        ],
    )
    def t(dist_hbm, out_hbm, rowbuf, cand_v, cand_i, outb):
        wid = lax.axis_index("s") * 2 + lax.axis_index("c")
        lane = lax.iota(jnp.int32, 16)

        def splat_min(v):
            for sh in (8, 4, 2, 1):
                v = jnp.minimum(v, _perm(v, jnp.bitwise_xor(lane, sh)))
            return v

        def splat_max(v):
            for sh in (8, 4, 2, 1):
                v = jnp.maximum(v, _perm(v, jnp.bitwise_xor(lane, sh)))
            return v

        def row_body(r, base_row):
            rb = r * N
            row_gid = base_row + r
            m0 = rowbuf[pl.ds(rb, 16)]
            m1 = rowbuf[pl.ds(rb + 512, 16)]
            for j in range(1, 32):
                m0 = jnp.minimum(m0, rowbuf[pl.ds(rb + j * 16, 16)])
                m1 = jnp.minimum(m1, rowbuf[pl.ds(rb + 512 + j * 16, 16)])
            thv = splat_max(jnp.maximum(m0, m1))
            ptr = jnp.int32(0)
            for j in range(64):
                v = rowbuf[pl.ds(rb + j * 16, 16)]
                msk = v <= thv
                plsc.store_compressed(cand_v.at[pl.ds(ptr, 16)], v, mask=msk)
                plsc.store_compressed(cand_i.at[pl.ds(ptr, 16)],
                                      lane + (j * 16), mask=msk)
                ptr = ptr + _cnt(msk)
            cand_v[pl.ds(ptr, 16)] = jnp.full((16,), INF)
            nt = (ptr + 15) // 16

            def rev(v):
                return lax.rev(v, (0,))

            def sort16(v):
                return lax.sort(v, dimension=0)

            def merge(tc, carry):
                b0, b1 = carry
                vs = sort16(cand_v[pl.ds(tc * 16, 16)])
                lo16 = sort16(jnp.minimum(b1, rev(vs)))
                b0n = sort16(jnp.minimum(b0, rev(lo16)))
                b1n = sort16(jnp.maximum(b0, rev(lo16)))
                return b0n, b1n

            b0, b1 = lax.fori_loop(0, nt, merge,
                                   (jnp.full((16,), INF), jnp.full((16,), INF)))
            tau = _perm(b1, jnp.full((16,), 3, jnp.int32))

            def cntA(tc, m1c):
                v = cand_v[pl.ds(tc * 16, 16)]
                return m1c + _cnt(v < tau)

            m1c = lax.fori_loop(0, nt, cntA, jnp.int32(0))
            m2v = jnp.full((16,), K - m1c, jnp.int32)

            gbase = jnp.full((16,), (row_gid // N) * N, jnp.int32)

            def emit(tc, carry):
                optr, eqseen = carry
                v = cand_v[pl.ds(tc * 16, 16)]
                ci = cand_i[pl.ds(tc * 16, 16)]
                lt = v < tau
                eq = v == tau
                eqc = eq.astype(jnp.int32)
                for sh in (1, 2, 4, 8):
                    shifted = _perm(eqc, jnp.maximum(lane - sh, 0))
                    eqc = eqc + jnp.where(lane >= sh, shifted, 0)
                eqc = eqc + eqseen
                em = lt | (eq & (eqc <= m2v))
                plsc.store_compressed(outb.at[pl.ds(r * K + optr, 16)],
                                      ci + gbase, mask=em)
                return (optr + _cnt(em),
                        eqseen + plsc.all_reduce_population_count(eq))

            lax.fori_loop(0, nt, emit,
                          (jnp.int32(0), jnp.zeros((16,), jnp.int32)))
            return base_row

        def g_body(g, _):
            base_row = wid * per_w + g * RG
            pltpu.sync_copy(dist_hbm.at[pl.ds(base_row * N, RG * N)], rowbuf)
            lax.fori_loop(0, RG, row_body, base_row)
            pltpu.sync_copy(outb.at[pl.ds(0, RG * K)],
                            out_hbm.at[pl.ds(base_row * K, RG * K)])
            return 0

        lax.fori_loop(0, n_g, g_body, 0)

    return t(dist.reshape(BN * N))


def _gather_nbr(h2d, idx_flat, C):
    NW = 32
    CHUNK = 128
    ROWS = BN * K
    per_w = ROWS // NW
    n_it = per_w // CHUNK
    mesh = plsc.VectorSubcoreMesh(core_axis_name="c", subcore_axis_name="s")

    @functools.partial(
        pl.kernel, mesh=mesh,
        compiler_params=pltpu.CompilerParams(use_tc_tiling_on_sc=False, needs_layout_passes=False),
        out_type=jax.ShapeDtypeStruct((ROWS, C), jnp.float32),
        scratch_types=[
            pltpu.VMEM((CHUNK,), jnp.int32),
            pltpu.VMEM((CHUNK,), jnp.int32),
            pltpu.VMEM((CHUNK, C), jnp.float32),
            pltpu.VMEM((CHUNK, C), jnp.float32),
            pltpu.SemaphoreType.DMA,
            pltpu.SemaphoreType.DMA,
        ],
    )
    def g(idx_hbm, table_hbm, out_hbm, idx0, idx1, rows0, rows1, sem0, sem1):
        wid = lax.axis_index("s") * 2 + lax.axis_index("c")
        bufs = ((idx0, rows0, sem0), (idx1, rows1, sem1))

        for par in (0, 1):
            ib, rb, sm = bufs[par]
            base = wid * per_w + par * CHUNK
            pltpu.sync_copy(idx_hbm.at[pl.ds(base, CHUNK)], ib)
            pltpu.async_copy(table_hbm.at[ib], rb, sm)

        def body(p, _):
            for par in (0, 1):
                ib, rb, sm = bufs[par]
                i = 2 * p + par
                base = wid * per_w + i * CHUNK
                pltpu.make_async_copy(table_hbm.at[ib], rb, sm).wait()
                pltpu.sync_copy(rb, out_hbm.at[pl.ds(base, CHUNK)])

                @pl.when(i + 2 < n_it)
                def _():
                    nbase = base + 2 * CHUNK
                    pltpu.sync_copy(idx_hbm.at[pl.ds(nbase, CHUNK)], ib)
                    pltpu.async_copy(table_hbm.at[ib], rb, sm)
            return 0

        lax.fori_loop(0, n_it // 2, body, 0)

    return g(idx_flat, h2d)


def _edge_conv_layer(h2d, Cin, tw, tb, pw, pb, g, bta):
    Cout = tw.shape[1]
    dist = _dist(h2d, Cin)
    idx24 = _topk_idx(dist)
    nbr = _gather_nbr(h2d, idx24, Cin)
    emax, emin, stats = _edge(nbr, h2d, tw, pw, (tb + pb).reshape(1, -1),
                              Cin, Cout)
    return _bnact(emax, emin, stats, g, bta, Cout)


def kernel(x,
           theta_W0, theta_b0, phi_W0, phi_b0, bn_g0, bn_b0,
           theta_W1, theta_b1, phi_W1, phi_b1, bn_g1, bn_b1,
           theta_W2, theta_b2, phi_W2, phi_b2, bn_g2, bn_b2,
           theta_W3, theta_b3, phi_W3, phi_b3, bn_g3, bn_b3,
           proj_W, proj_b,
           emb_W0, emb_b0, emb_W1, emb_b1,
           out_W, out_b):
    inp = dict(locals())
    h = jnp.pad(x.reshape(BN, IN_DIMS), ((0, 0), (0, 5)))
    tw0 = jnp.pad(theta_W0, ((0, 5), (0, 0)))
    pw0 = jnp.pad(phi_W0, ((0, 5), (0, 0)))
    Cin = 8
    hs = []
    for i in range(len(FEATURE_DIMS)):
        tw = tw0 if i == 0 else inp[f"theta_W{i}"]
        pw = pw0 if i == 0 else inp[f"phi_W{i}"]
        h = _edge_conv_layer(h, Cin, tw, inp[f"theta_b{i}"],
                             pw, inp[f"phi_b{i}"],
                             inp[f"bn_g{i}"], inp[f"bn_b{i}"])
        Cin = FEATURE_DIMS[i]
        hs.append(h)
    h2 = _proj_pool(hs, proj_W, proj_b)
    return _head(h2, emb_W0, emb_b0, emb_W1, emb_b1, out_W, out_b)


# speedup vs baseline: 8.8215x; 1.0321x over previous
import functools
import jax
import jax.numpy as jnp
from jax import lax
from jax.experimental import pallas as pl
from jax.experimental.pallas import tpu as pltpu
from jax.experimental.pallas import tpu_sc as plsc

K = 20
KP = 24
B, N, IN_DIMS = 16, 1024, 3
FEATURE_DIMS = [64, 64, 128, 256]
BN = B * N


def _leaky(x):
    return jnp.where(x >= 0, x, 0.2 * x)


def _bf(x):
    return x.astype(jnp.bfloat16)


def _dist_body(h_ref, dist_ref):
    h = h_ref[...]
    sq = jnp.sum(h * h, axis=1, keepdims=True)
    hb = _bf(h)
    mm = lax.dot_general(hb, hb, (((1,), (1,)), ((), ())),
                         preferred_element_type=jnp.float32)
    dist_ref[...] = (sq + sq.reshape(1, N)) - 2.0 * mm


def _dist(h2d, C):
    return pl.pallas_call(
        _dist_body,
        grid=(B,),
        in_specs=[pl.BlockSpec((N, C), lambda b: (b, 0))],
        out_specs=pl.BlockSpec((N, N), lambda b: (b, 0)),
        out_shape=jax.ShapeDtypeStruct((BN, N), jnp.float32),
    )(h2d)


def _edge_body(nbr_ref, h_ref, tw_ref, pw_ref, bias_ref,
               emax_ref, emin_ref, stats_ref, acc_ref, DSTBLK, Cin, Cout):
    i = pl.program_id(0)
    nbr = nbr_ref[...].reshape(DSTBLK, K, Cin)
    h = h_ref[...]
    diff = h[:, None, :] - nbr
    db = _bf(diff).reshape(DSTBLK * K, Cin)
    nb = _bf(nbr).reshape(DSTBLK * K, Cin)
    e = (jnp.dot(db, tw_ref[...], preferred_element_type=jnp.float32)
         + jnp.dot(nb, pw_ref[...], preferred_element_type=jnp.float32)
         + bias_ref[...])
    e = e.reshape(DSTBLK, K, Cout)
    emax_ref[...] = jnp.max(e, axis=1)
    emin_ref[...] = jnp.min(e, axis=1)
    s = jnp.sum(e.reshape(DSTBLK * K, Cout), axis=0, keepdims=True)
    s2 = jnp.sum((e * e).reshape(DSTBLK * K, Cout), axis=0, keepdims=True)

    @pl.when(i == 0)
    def _():
        acc_ref[...] = jnp.zeros_like(acc_ref)

    acc_ref[0:1, :] += s
    acc_ref[1:2, :] += s2
    stats_ref[...] = acc_ref[...]


def _edge(nbr2d, h2d, tw, pw, bias, Cin, Cout):
    DSTBLK = 256
    body = functools.partial(_edge_body, DSTBLK=DSTBLK, Cin=Cin, Cout=Cout)
    return pl.pallas_call(
        body,
        grid=(BN // DSTBLK,),
        in_specs=[
            pl.BlockSpec((DSTBLK * K, Cin), lambda i: (i, 0)),
            pl.BlockSpec((DSTBLK, Cin), lambda i: (i, 0)),
            pl.BlockSpec((Cin, Cout), lambda i: (0, 0)),
            pl.BlockSpec((Cin, Cout), lambda i: (0, 0)),
            pl.BlockSpec((1, Cout), lambda i: (0, 0)),
        ],
        out_specs=[
            pl.BlockSpec((DSTBLK, Cout), lambda i: (i, 0)),
            pl.BlockSpec((DSTBLK, Cout), lambda i: (i, 0)),
            pl.BlockSpec((8, Cout), lambda i: (0, 0)),
        ],
        out_shape=[
            jax.ShapeDtypeStruct((BN, Cout), jnp.float32),
            jax.ShapeDtypeStruct((BN, Cout), jnp.float32),
            jax.ShapeDtypeStruct((8, Cout), jnp.float32),
        ],
        scratch_shapes=[pltpu.VMEM((8, Cout), jnp.float32)],
    )(nbr2d, h2d, _bf(tw), _bf(pw), bias)


def _bnact_body(emax_ref, emin_ref, stats_ref, g_ref, bta_ref, out_ref):
    cnt = float(BN * K)
    mean = stats_ref[0:1, :] / cnt
    var = stats_ref[1:2, :] / cnt - mean * mean
    scale = g_ref[...] * lax.rsqrt(var + 1e-5)
    red = jnp.where(scale >= 0, emax_ref[...], emin_ref[...])
    out_ref[...] = _leaky(scale * (red - mean) + bta_ref[...])


def _bnact(emax, emin, stats, g, bta, Cout):
    RB = 1024
    return pl.pallas_call(
        _bnact_body,
        grid=(BN // RB,),
        in_specs=[
            pl.BlockSpec((RB, Cout), lambda i: (i, 0)),
            pl.BlockSpec((RB, Cout), lambda i: (i, 0)),
            pl.BlockSpec((8, Cout), lambda i: (0, 0)),
            pl.BlockSpec((1, Cout), lambda i: (0, 0)),
            pl.BlockSpec((1, Cout), lambda i: (0, 0)),
        ],
        out_specs=pl.BlockSpec((RB, Cout), lambda i: (i, 0)),
        out_shape=jax.ShapeDtypeStruct((BN, Cout), jnp.float32),
    )(emax, emin, stats, g.reshape(1, -1), bta.reshape(1, -1))


def _proj_body(h0_ref, h1_ref, h2_ref, h3_ref, w_ref, b_ref, out_ref):
    hcat = jnp.concatenate([h0_ref[...], h1_ref[...], h2_ref[...],
                            h3_ref[...]], axis=1)
    p = (jnp.dot(_bf(hcat), w_ref[...], preferred_element_type=jnp.float32)
         + b_ref[...])
    mx = jnp.max(p, axis=0, keepdims=True)
    mean = jnp.sum(p, axis=0, keepdims=True) / float(N)
    out_ref[0, 0:1, :] = mx
    out_ref[0, 1:2, :] = mean


def _proj_pool(hs, w, b):
    E = w.shape[1]
    out = pl.pallas_call(
        _proj_body,
        grid=(B,),
        in_specs=[pl.BlockSpec((1024, c), lambda i: (i, 0))
                  for c in FEATURE_DIMS]
        + [pl.BlockSpec((sum(FEATURE_DIMS), E), lambda i: (0, 0)),
           pl.BlockSpec((1, E), lambda i: (0, 0))],
        out_specs=pl.BlockSpec((1, 2, E), lambda i: (i, 0, 0)),
        out_shape=jax.ShapeDtypeStruct((B, 2, E), jnp.float32),
    )(*hs, _bf(w), b.reshape(1, -1))
    return jnp.concatenate([out[:, 0, :], out[:, 1, :]], axis=1)


def _head_body(h_ref, w0_ref, b0_ref, w1_ref, b1_ref, w2_ref, b2_ref, o_ref):
    h = h_ref[...]
    h = _leaky(jnp.dot(_bf(h), w0_ref[...],
                       preferred_element_type=jnp.float32) + b0_ref[...])
    h = _leaky(jnp.dot(_bf(h), w1_ref[...],
                       preferred_element_type=jnp.float32) + b1_ref[...])
    o_ref[...] = (jnp.dot(_bf(h), w2_ref[...],
                          preferred_element_type=jnp.float32) + b2_ref[...])


def _head(h2, w0, b0, w1, b1, w2, b2):
    return pl.pallas_call(
        _head_body,
        out_shape=jax.ShapeDtypeStruct((B, w2.shape[1]), jnp.float32),
    )(h2, _bf(w0), b0.reshape(1, -1), _bf(w1), b1.reshape(1, -1),
      _bf(w2), b2.reshape(1, -1))


def _topk_idx(dist):
    NW = 32
    RG = 16
    per_w = BN // NW
    n_g = per_w // RG
    INF = jnp.float32(jnp.inf)
    mesh = plsc.VectorSubcoreMesh(core_axis_name="c", subcore_axis_name="s")

    def _scal(v, lane_i):
        return lax.squeeze(lax.slice(v, (lane_i,), (lane_i + 1,)), (0,))

    def _cnt(msk):
        return _scal(plsc.all_reduce_population_count(msk), 0)

    _DN = lax.GatherDimensionNumbers(offset_dims=(), collapsed_slice_dims=(0,),
                                     start_index_map=(0,))

    def _perm(v, idxvec):
        return lax.gather(v, idxvec[:, None], _DN, (1,),
                          mode=lax.GatherScatterMode.PROMISE_IN_BOUNDS)

    @functools.partial(
        pl.kernel, mesh=mesh,
        compiler_params=pltpu.CompilerParams(use_tc_tiling_on_sc=False, needs_layout_passes=False),
        out_type=jax.ShapeDtypeStruct((BN * K,), jnp.int32),
        scratch_types=[
            pltpu.VMEM((RG * N,), jnp.float32),
            pltpu.VMEM((N + 16,), jnp.float32),
            pltpu.VMEM((N + 16,), jnp.int32),
            pltpu.VMEM((RG * K + 16,), jnp.int32),
        ],
    )
    def t(dist_hbm, out_hbm, rowbuf, cand_v, cand_i, outb):
        wid = lax.axis_index("s") * 2 + lax.axis_index("c")
        lane = lax.iota(jnp.int32, 16)

        def splat_min(v):
            for sh in (8, 4, 2, 1):
                v = jnp.minimum(v, _perm(v, jnp.bitwise_xor(lane, sh)))
            return v

        def splat_max(v):
            for sh in (8, 4, 2, 1):
                v = jnp.maximum(v, _perm(v, jnp.bitwise_xor(lane, sh)))
            return v

        def row_body(r, base_row):
            rb = r * N
            row_gid = base_row + r
            m0 = rowbuf[pl.ds(rb, 16)]
            m1 = rowbuf[pl.ds(rb + 512, 16)]
            for j in range(1, 32):
                m0 = jnp.minimum(m0, rowbuf[pl.ds(rb + j * 16, 16)])
                m1 = jnp.minimum(m1, rowbuf[pl.ds(rb + 512 + j * 16, 16)])
            m1m = jnp.where(lane < 4, m1, jnp.float32(-jnp.inf))
            thv = jnp.minimum(splat_max(jnp.maximum(m0, m1)),
                              jnp.maximum(splat_max(m0), splat_max(m1m)))
            ptr = jnp.int32(0)
            for j in range(64):
                v = rowbuf[pl.ds(rb + j * 16, 16)]
                msk = v <= thv
                plsc.store_compressed(cand_v.at[pl.ds(ptr, 16)], v, mask=msk)
                plsc.store_compressed(cand_i.at[pl.ds(ptr, 16)],
                                      lane + (j * 16), mask=msk)
                ptr = ptr + _cnt(msk)
            cand_v[pl.ds(ptr, 16)] = jnp.full((16,), INF)
            nt = (ptr + 15) // 16

            def rev(v):
                return lax.rev(v, (0,))

            def sort16(v):
                return lax.sort(v, dimension=0)

            def merge(tc, carry):
                b0, b1 = carry
                vs = sort16(cand_v[pl.ds(tc * 16, 16)])
                lo16 = sort16(jnp.minimum(b1, rev(vs)))
                b0n = sort16(jnp.minimum(b0, rev(lo16)))
                b1n = sort16(jnp.maximum(b0, rev(lo16)))
                return b0n, b1n

            b0, b1 = lax.fori_loop(0, nt, merge,
                                   (jnp.full((16,), INF), jnp.full((16,), INF)))
            tau = _perm(b1, jnp.full((16,), 3, jnp.int32))
            m1c = _cnt(b0 < tau) + _cnt(b1 < tau)
            m2v = jnp.full((16,), K - m1c, jnp.int32)

            gbase = jnp.full((16,), (row_gid // N) * N, jnp.int32)

            def emit(tc, carry):
                optr, eqseen = carry
                v = cand_v[pl.ds(tc * 16, 16)]
                ci = cand_i[pl.ds(tc * 16, 16)]
                lt = v < tau
                eq = v == tau
                eqc = eq.astype(jnp.int32)
                for sh in (1, 2, 4, 8):
                    shifted = _perm(eqc, jnp.maximum(lane - sh, 0))
                    eqc = eqc + jnp.where(lane >= sh, shifted, 0)
                eqc = eqc + eqseen
                em = lt | (eq & (eqc <= m2v))
                plsc.store_compressed(outb.at[pl.ds(r * K + optr, 16)],
                                      ci + gbase, mask=em)
                return (optr + _cnt(em),
                        eqseen + plsc.all_reduce_population_count(eq))

            lax.fori_loop(0, nt, emit,
                          (jnp.int32(0), jnp.zeros((16,), jnp.int32)))
            return base_row

        def g_body(g, _):
            base_row = wid * per_w + g * RG
            pltpu.sync_copy(dist_hbm.at[pl.ds(base_row * N, RG * N)], rowbuf)
            lax.fori_loop(0, RG, row_body, base_row)
            pltpu.sync_copy(outb.at[pl.ds(0, RG * K)],
                            out_hbm.at[pl.ds(base_row * K, RG * K)])
            return 0

        lax.fori_loop(0, n_g, g_body, 0)

    return t(dist.reshape(BN * N))


def _gather_nbr(h2d, idx_flat, C):
    NW = 32
    CHUNK = 128
    ROWS = BN * K
    per_w = ROWS // NW
    n_it = per_w // CHUNK
    mesh = plsc.VectorSubcoreMesh(core_axis_name="c", subcore_axis_name="s")

    @functools.partial(
        pl.kernel, mesh=mesh,
        compiler_params=pltpu.CompilerParams(use_tc_tiling_on_sc=False, needs_layout_passes=False),
        out_type=jax.ShapeDtypeStruct((ROWS, C), jnp.float32),
        scratch_types=[
            pltpu.VMEM((CHUNK,), jnp.int32),
            pltpu.VMEM((CHUNK,), jnp.int32),
            pltpu.VMEM((CHUNK, C), jnp.float32),
            pltpu.VMEM((CHUNK, C), jnp.float32),
            pltpu.SemaphoreType.DMA,
            pltpu.SemaphoreType.DMA,
        ],
    )
    def g(idx_hbm, table_hbm, out_hbm, idx0, idx1, rows0, rows1, sem0, sem1):
        wid = lax.axis_index("s") * 2 + lax.axis_index("c")
        bufs = ((idx0, rows0, sem0), (idx1, rows1, sem1))

        for par in (0, 1):
            ib, rb, sm = bufs[par]
            base = wid * per_w + par * CHUNK
            pltpu.sync_copy(idx_hbm.at[pl.ds(base, CHUNK)], ib)
            pltpu.async_copy(table_hbm.at[ib], rb, sm)

        def body(p, _):
            for par in (0, 1):
                ib, rb, sm = bufs[par]
                i = 2 * p + par
                base = wid * per_w + i * CHUNK
                pltpu.make_async_copy(table_hbm.at[ib], rb, sm).wait()
                pltpu.sync_copy(rb, out_hbm.at[pl.ds(base, CHUNK)])

                @pl.when(i + 2 < n_it)
                def _():
                    nbase = base + 2 * CHUNK
                    pltpu.sync_copy(idx_hbm.at[pl.ds(nbase, CHUNK)], ib)
                    pltpu.async_copy(table_hbm.at[ib], rb, sm)
            return 0

        lax.fori_loop(0, n_it // 2, body, 0)

    return g(idx_flat, h2d)


def _edge_conv_layer(h2d, Cin, tw, tb, pw, pb, g, bta):
    Cout = tw.shape[1]
    dist = _dist(h2d, Cin)
    idx24 = _topk_idx(dist)
    nbr = _gather_nbr(h2d, idx24, Cin)
    emax, emin, stats = _edge(nbr, h2d, tw, pw, (tb + pb).reshape(1, -1),
                              Cin, Cout)
    return _bnact(emax, emin, stats, g, bta, Cout)


def kernel(x,
           theta_W0, theta_b0, phi_W0, phi_b0, bn_g0, bn_b0,
           theta_W1, theta_b1, phi_W1, phi_b1, bn_g1, bn_b1,
           theta_W2, theta_b2, phi_W2, phi_b2, bn_g2, bn_b2,
           theta_W3, theta_b3, phi_W3, phi_b3, bn_g3, bn_b3,
           proj_W, proj_b,
           emb_W0, emb_b0, emb_W1, emb_b1,
           out_W, out_b):
    inp = dict(locals())
    h = jnp.pad(x.reshape(BN, IN_DIMS), ((0, 0), (0, 5)))
    tw0 = jnp.pad(theta_W0, ((0, 5), (0, 0)))
    pw0 = jnp.pad(phi_W0, ((0, 5), (0, 0)))
    Cin = 8
    hs = []
    for i in range(len(FEATURE_DIMS)):
        tw = tw0 if i == 0 else inp[f"theta_W{i}"]
        pw = pw0 if i == 0 else inp[f"phi_W{i}"]
        h = _edge_conv_layer(h, Cin, tw, inp[f"theta_b{i}"],
                             pw, inp[f"phi_b{i}"],
                             inp[f"bn_g{i}"], inp[f"bn_b{i}"])
        Cin = FEATURE_DIMS[i]
        hs.append(h)
    h2 = _proj_pool(hs, proj_W, proj_b)
    return _head(h2, emb_W0, emb_b0, emb_W1, emb_b1, out_W, out_b)
